# baseline (speedup 1.0000x reference)
import functools

import jax
import jax.numpy as jnp
from jax import lax
from jax.experimental import pallas as pl
from jax.experimental.pallas import tpu as pltpu
from jax.experimental.pallas import tpu_sc as plsc

N = 10000
E = 320000
D = 128
NG = 128
NEG = 0.2

NC, NS, L = 2, 16, 16
NW = NC * NS
NP = 10240
K = 128
NCHUNK = 88
EPT = NCHUNK * K
E2P = NW * EPT
SL = NP // NS
NSS = NCHUNK // 8


def _leaky(v):
    return jnp.maximum(v, NEG * v)


def _gmax_of(as_v):
    def body(i, acc):
        return jnp.maximum(acc, as_v[pl.ds(i * L, L)])
    m = lax.fori_loop(0, NP // L, body, jnp.full((L,), -jnp.inf, jnp.float32))
    return jnp.max(m)


def _phase_a_body(src_hbm, dst_hbm, as_hbm, ad_hbm, den_hbm, ex_hbm,
                  as_v, ad_v, srcv, dstv, exv, dloc, accv, tmp2, dsh, sem):
    cid = lax.axis_index("c")
    sid = lax.axis_index("s")
    wid = cid * NS + sid
    pltpu.sync_copy(as_hbm, as_v)
    pltpu.sync_copy(ad_hbm, ad_v)
    pltpu.sync_copy(src_hbm.at[pl.ds(wid * EPT, EPT)], srcv)
    pltpu.sync_copy(dst_hbm.at[pl.ds(wid * EPT, EPT)], dstv)

    def zero(i, _):
        dloc[pl.ds(i * L, L)] = jnp.zeros((L,), jnp.float32)
        return _
    lax.fori_loop(0, NP // L, zero, None)

    gmax = _gmax_of(as_v)

    def edge(i, _):
        s = srcv[pl.ds(i * L, L)]
        d = dstv[pl.ds(i * L, L)]
        a1 = plsc.load_gather(as_v, [s])
        a2 = plsc.load_gather(ad_v, [d])
        e = _leaky(a1 + a2)
        m = _leaky(gmax + a2)
        ex = jnp.exp(e - m)
        exv[pl.ds(i * L, L)] = ex
        plsc.addupdate_scatter(dloc, [d], ex)
        return _
    lax.fori_loop(0, EPT // L, edge, None)
    pltpu.sync_copy(exv, ex_hbm.at[pl.ds(wid * EPT, EPT)])

    pltpu.sync_copy(dloc, dsh.at[sid])
    plsc.subcore_barrier()
    pltpu.sync_copy(dsh.at[:, pl.ds(sid * SL, SL)], tmp2)

    def macc(i, _):
        tot = tmp2[0, pl.ds(i * L, L)]
        for t in range(1, NS):
            tot = tot + tmp2[t, pl.ds(i * L, L)]
        accv[pl.ds(i * L, L)] = tot
        return _
    lax.fori_loop(0, SL // L, macc, None)
    pltpu.sync_copy(accv, den_hbm.at[cid, pl.ds(sid * SL, SL)])


_SC_PARAMS = pltpu.CompilerParams(needs_layout_passes=False)

_phase_a = functools.partial(
    pl.kernel,
    out_type=(jax.ShapeDtypeStruct((NC, NP), jnp.float32),
              jax.ShapeDtypeStruct((E2P,), jnp.float32)),
    mesh=plsc.VectorSubcoreMesh(core_axis_name="c", subcore_axis_name="s"),
    compiler_params=_SC_PARAMS,
    scratch_types=[
        pltpu.VMEM((NP,), jnp.float32),
        pltpu.VMEM((NP,), jnp.float32),
        pltpu.VMEM((EPT,), jnp.int32),
        pltpu.VMEM((EPT,), jnp.int32),
        pltpu.VMEM((EPT,), jnp.float32),
        pltpu.VMEM((NP,), jnp.float32),
        pltpu.VMEM((SL,), jnp.float32),
        pltpu.VMEM((NS, SL), jnp.float32),
        pltpu.VMEM_SHARED((NS, NP), jnp.float32),
        pltpu.SemaphoreType.DMA,
    ],
)(_phase_a_body)


def _phase_b_body(h_hbm, src_hbm, dst_hbm, ex_hbm, out_hbm,
                  srcc, dstc, exc, gidx, sidx, rows, acc_sh,
                  msem, gsem, ssem):
    cid = lax.axis_index("c")
    sid = lax.axis_index("s")
    wid = cid * NS + sid
    tbase = wid * EPT

    def zrow(r, _):
        ridx = jnp.full((L,), r, jnp.int32)
        for c in range(D // L):
            cidx = c * L + lax.iota(jnp.int32, L)
            plsc.store_scatter(rows.at[0], [ridx, cidx],
                               jnp.zeros((L,), jnp.float32))
        return _
    lax.fori_loop(0, K, zrow, None)
    for z in range(SL // K):
        pltpu.sync_copy(rows.at[0], acc_sh.at[pl.ds(sid * SL + z * K, K)])
    plsc.subcore_barrier()

    def stage_meta(j, slot):
        base = tbase + j * K
        pltpu.async_copy(src_hbm.at[pl.ds(base, K)], srcc.at[slot],
                         msem.at[slot])
        pltpu.async_copy(dst_hbm.at[pl.ds(base, K)], dstc.at[slot],
                         msem.at[slot])
        pltpu.async_copy(ex_hbm.at[pl.ds(base, K)], exc.at[slot],
                         msem.at[slot])

    def wait_meta(slot):
        pltpu.make_async_copy(src_hbm.at[pl.ds(0, K)], srcc.at[slot],
                              msem.at[slot]).wait()
        pltpu.make_async_copy(dst_hbm.at[pl.ds(0, K)], dstc.at[slot],
                              msem.at[slot]).wait()
        pltpu.make_async_copy(ex_hbm.at[pl.ds(0, K)], exc.at[slot],
                              msem.at[slot]).wait()

    def copy_idx(src2d, mslot, dst2d, rslot):
        for u in range(K // L):
            dst2d[rslot, pl.ds(u * L, L)] = src2d[mslot, pl.ds(u * L, L)]

    def start_gather(rslot):
        pltpu.async_copy(h_hbm.at[gidx.at[rslot]], rows.at[rslot],
                         gsem.at[rslot])

    def wait_gather(rslot):
        pltpu.make_async_copy(h_hbm.at[gidx.at[rslot]], rows.at[rslot],
                              gsem.at[rslot]).wait()

    def start_scatter(rslot):
        pltpu.async_copy(rows.at[rslot], acc_sh.at[sidx.at[rslot]],
                         ssem.at[rslot], add=True)

    def wait_scatter(rslot):
        pltpu.make_async_copy(rows.at[rslot], acc_sh.at[sidx.at[rslot]],
                              ssem.at[rslot]).wait()

    def scale(mslot, rslot):
        def body(r4, _):
            for k in range(4):
                ridx = jnp.full((L,), r4 * 4 + k, jnp.int32)
                wb = plsc.load_gather(exc.at[mslot], [ridx])
                for c in range(D // L):
                    cidx = c * L + lax.iota(jnp.int32, L)
                    v = plsc.load_gather(rows.at[rslot], [ridx, cidx])
                    plsc.store_scatter(rows.at[rslot], [ridx, cidx], v * wb)
            return _
        lax.fori_loop(0, K // 4, body, None)

    def section(s, b, first):
        j = 8 * s + b
        rs, rn = b % 2, (b + 1) % 2
        mn = (b + 1) % 8
        wait_gather(rs)
        wb_slot = b
        scale(wb_slot, rs)
        copy_idx(dstc, wb_slot, sidx, rs)
        start_scatter(rs)
        if not first:
            wait_scatter(rn)
        wait_meta(mn)
        copy_idx(srcc, mn, gidx, rn)
        start_gather(rn)
        stage_meta(jnp.minimum(j + 8, NCHUNK - 1), wb_slot)

    for b in range(8):
        stage_meta(b, b)
    wait_meta(0)
    copy_idx(srcc, 0, gidx, 0)
    start_gather(0)

    for b in range(8):
        section(0, b, first=(b == 0))

    def superstep(s, _):
        for b in range(8):
            section(s, b, first=False)
        return _
    lax.fori_loop(1, NSS, superstep, None)

    wait_gather(0)
    wait_scatter(1)
    for b in range(1, 8):
        wait_meta(b)

    plsc.subcore_barrier()
    pltpu.sync_copy(acc_sh.at[pl.ds(sid * SL, SL)],
                    out_hbm.at[cid, pl.ds(sid * SL, SL)])


_phase_b = functools.partial(
    pl.kernel,
    out_type=jax.ShapeDtypeStruct((NC, NP, D), jnp.float32),
    mesh=plsc.VectorSubcoreMesh(core_axis_name="c", subcore_axis_name="s"),
    compiler_params=_SC_PARAMS,
    scratch_types=[
        pltpu.VMEM((8, K), jnp.int32),
        pltpu.VMEM((8, K), jnp.int32),
        pltpu.VMEM((8, K), jnp.float32),
        pltpu.VMEM((2, K), jnp.int32),
        pltpu.VMEM((2, K), jnp.int32),
        pltpu.VMEM((2, K, D), jnp.float32),
        pltpu.VMEM_SHARED((NP, D), jnp.float32),
        pltpu.SemaphoreType.DMA((8,)),
        pltpu.SemaphoreType.DMA((2,)),
        pltpu.SemaphoreType.DMA((2,)),
    ],
)(_phase_b_body)


_RB = 512


def _dense1_body(x_ref, w_ref, asr_ref, adr_ref, h_ref, as_ref, ad_ref):
    h = jnp.dot(x_ref[...], w_ref[...], preferred_element_type=jnp.float32)
    h_ref[...] = h
    as_ref[...] = jnp.sum(h * asr_ref[...], axis=1, keepdims=True)
    ad_ref[...] = jnp.sum(h * adr_ref[...], axis=1, keepdims=True)


def _dense2_body(p_ref, den_ref, b_ref, w_ref, asr_ref, adr_ref,
                 h_ref, as_ref, ad_ref):
    i = pl.program_id(0)
    rows = i * _RB + lax.broadcasted_iota(jnp.int32, (_RB, D), 0)
    den = den_ref[0] + den_ref[1] + 1e-16
    x = (p_ref[0] + p_ref[1]) / den + b_ref[...]
    x = jnp.where(rows < N, x, 0.0)
    h = jnp.dot(x, w_ref[...], preferred_element_type=jnp.float32)
    h_ref[...] = h
    as_ref[...] = jnp.sum(h * asr_ref[...], axis=1, keepdims=True)
    ad_ref[...] = jnp.sum(h * adr_ref[...], axis=1, keepdims=True)


def _dense1(x, w, a_src, a_dst):
    return pl.pallas_call(
        _dense1_body,
        grid=(NP // _RB,),
        in_specs=[
            pl.BlockSpec((_RB, D), lambda i: (i, 0)),
            pl.BlockSpec((D, D), lambda i: (0, 0)),
            pl.BlockSpec((1, D), lambda i: (0, 0)),
            pl.BlockSpec((1, D), lambda i: (0, 0)),
        ],
        out_specs=[
            pl.BlockSpec((_RB, D), lambda i: (i, 0)),
            pl.BlockSpec((_RB, 1), lambda i: (i, 0)),
            pl.BlockSpec((_RB, 1), lambda i: (i, 0)),
        ],
        out_shape=[
            jax.ShapeDtypeStruct((NP, D), jnp.float32),
            jax.ShapeDtypeStruct((NP, 1), jnp.float32),
            jax.ShapeDtypeStruct((NP, 1), jnp.float32),
        ],
    )(x, w, a_src.reshape(1, D), a_dst.reshape(1, D))


def _dense2(p, den, b, w, a_src, a_dst):
    return pl.pallas_call(
        _dense2_body,
        grid=(NP // _RB,),
        in_specs=[
            pl.BlockSpec((NC, _RB, D), lambda i: (0, i, 0)),
            pl.BlockSpec((NC, _RB, 1), lambda i: (0, i, 0)),
            pl.BlockSpec((1, D), lambda i: (0, 0)),
            pl.BlockSpec((D, D), lambda i: (0, 0)),
            pl.BlockSpec((1, D), lambda i: (0, 0)),
            pl.BlockSpec((1, D), lambda i: (0, 0)),
        ],
        out_specs=[
            pl.BlockSpec((_RB, D), lambda i: (i, 0)),
            pl.BlockSpec((_RB, 1), lambda i: (i, 0)),
            pl.BlockSpec((_RB, 1), lambda i: (i, 0)),
        ],
        out_shape=[
            jax.ShapeDtypeStruct((NP, D), jnp.float32),
            jax.ShapeDtypeStruct((NP, 1), jnp.float32),
            jax.ShapeDtypeStruct((NP, 1), jnp.float32),
        ],
    )(p, den.reshape(NC, NP, 1), b.reshape(1, D), w,
      a_src.reshape(1, D), a_dst.reshape(1, D))


_PB = 400


def _pool_body(p_ref, den_ref, b_ref, batch_ref, out_ref, acc, cnt):
    i = pl.program_id(0)
    den = den_ref[0] + den_ref[1] + 1e-16
    x = (p_ref[0] + p_ref[1]) / den + b_ref[...]
    onehot = (batch_ref[...] ==
              lax.broadcasted_iota(jnp.int32, (_PB, NG), 1)).astype(jnp.float32)
    psum = lax.dot_general(onehot, x, (((0,), (0,)), ((), ())),
                           preferred_element_type=jnp.float32)
    pcnt = lax.dot_general(onehot, jnp.ones((_PB, 1), jnp.float32),
                           (((0,), (0,)), ((), ())),
                           preferred_element_type=jnp.float32)

    @pl.when(i == 0)
    def _():
        acc[...] = jnp.zeros_like(acc)
        cnt[...] = jnp.zeros_like(cnt)

    acc[...] += psum
    cnt[...] += pcnt

    @pl.when(i == N // _PB - 1)
    def _():
        out_ref[...] = acc[...] / jnp.maximum(cnt[...], 1.0)


def _pool(p, den, b, batch):
    return pl.pallas_call(
        _pool_body,
        grid=(N // _PB,),
        in_specs=[
            pl.BlockSpec((NC, _PB, D), lambda i: (0, i, 0)),
            pl.BlockSpec((NC, _PB, 1), lambda i: (0, i, 0)),
            pl.BlockSpec((1, D), lambda i: (0, 0)),
            pl.BlockSpec((_PB, 1), lambda i: (i, 0)),
        ],
        out_specs=pl.BlockSpec((NG, D), lambda i: (0, 0)),
        out_shape=jax.ShapeDtypeStruct((NG, D), jnp.float32),
        scratch_shapes=[
            pltpu.VMEM((NG, D), jnp.float32),
            pltpu.VMEM((NG, 1), jnp.float32),
        ],
    )(p, den.reshape(NC, NP, 1), b.reshape(1, D), batch.reshape(N, 1))


def kernel(x, edge_index, batch,
           W1, a_src1, a_dst1, b1, W2, a_src2, a_dst2, b2,
           W3, a_src3, a_dst3, b3):
    loop = jnp.arange(N, dtype=jnp.int32)
    padi = jnp.full((E2P - E - N,), N, jnp.int32)
    src = jnp.concatenate([edge_index[0], loop, padi])
    dst = jnp.concatenate([edge_index[1], loop, padi])
    xp = jnp.pad(x, ((0, NP - N), (0, 0)))

    h, asv, adv = _dense1(xp, W1, a_src1, a_dst1)
    for (w, a_s, a_d, b) in ((W2, a_src2, a_dst2, b1),
                             (W3, a_src3, a_dst3, b2)):
        den, ex = _phase_a(src, dst, asv.reshape(NP), adv.reshape(NP))
        p = _phase_b(h, src, dst, ex)
        h, asv, adv = _dense2(p, den, b, w, a_s, a_d)
    den, ex = _phase_a(src, dst, asv.reshape(NP), adv.reshape(NP))
    p = _phase_b(h, src, dst, ex)
    return _pool(p, den, b3, batch)


# speedup vs baseline: 2.0733x; 2.0733x over previous
import functools

import jax
import jax.numpy as jnp
from jax import lax
from jax.experimental import pallas as pl
from jax.experimental.pallas import tpu as pltpu
from jax.experimental.pallas import tpu_sc as plsc

N = 10000
E = 320000
D = 128
NG = 128
NEG = 0.2

NC, NS, L = 2, 16, 16
NW = NC * NS
NP = 10240
K = 128
NCHUNK = 88
EPT = NCHUNK * K
E2P = NW * EPT
SL = NP // NS
NSS = NCHUNK // 8


def _leaky(v):
    return jnp.maximum(v, NEG * v)


def _gmax_of(as_v):
    def body(i, acc):
        return jnp.maximum(acc, as_v[pl.ds(i * L, L)])
    m = lax.fori_loop(0, NP // L, body, jnp.full((L,), -jnp.inf, jnp.float32))
    return jnp.max(m)


def _phase_a_body(src_hbm, dst_hbm, as_hbm, ad_hbm, den_hbm, ex_hbm,
                  as_v, ad_v, srcv, dstv, exv, dloc, accv, tmp2, dsh, sem):
    cid = lax.axis_index("c")
    sid = lax.axis_index("s")
    wid = cid * NS + sid
    pltpu.sync_copy(as_hbm, as_v)
    pltpu.sync_copy(ad_hbm, ad_v)
    pltpu.sync_copy(src_hbm.at[pl.ds(wid * EPT, EPT)], srcv)
    pltpu.sync_copy(dst_hbm.at[pl.ds(wid * EPT, EPT)], dstv)

    def zero(i, _):
        dloc[pl.ds(i * L, L)] = jnp.zeros((L,), jnp.float32)
        return _
    lax.fori_loop(0, NP // L, zero, None)

    gmax = _gmax_of(as_v)

    def edge(i, _):
        s = srcv[pl.ds(i * L, L)]
        d = dstv[pl.ds(i * L, L)]
        a1 = plsc.load_gather(as_v, [s])
        a2 = plsc.load_gather(ad_v, [d])
        e = _leaky(a1 + a2)
        m = _leaky(gmax + a2)
        ex = jnp.exp(e - m)
        exv[pl.ds(i * L, L)] = ex
        plsc.addupdate_scatter(dloc, [d], ex)
        return _
    lax.fori_loop(0, EPT // L, edge, None)
    pltpu.sync_copy(exv, ex_hbm.at[pl.ds(wid * EPT, EPT)])

    pltpu.sync_copy(dloc, dsh.at[sid])
    plsc.subcore_barrier()
    pltpu.sync_copy(dsh.at[:, pl.ds(sid * SL, SL)], tmp2)

    def macc(i, _):
        tot = tmp2[0, pl.ds(i * L, L)]
        for t in range(1, NS):
            tot = tot + tmp2[t, pl.ds(i * L, L)]
        accv[pl.ds(i * L, L)] = tot
        return _
    lax.fori_loop(0, SL // L, macc, None)
    pltpu.sync_copy(accv, den_hbm.at[cid, pl.ds(sid * SL, SL)])


_SC_PARAMS = pltpu.CompilerParams(needs_layout_passes=False)

_phase_a = functools.partial(
    pl.kernel,
    out_type=(jax.ShapeDtypeStruct((NC, NP), jnp.float32),
              jax.ShapeDtypeStruct((E2P,), jnp.float32)),
    mesh=plsc.VectorSubcoreMesh(core_axis_name="c", subcore_axis_name="s"),
    compiler_params=_SC_PARAMS,
    scratch_types=[
        pltpu.VMEM((NP,), jnp.float32),
        pltpu.VMEM((NP,), jnp.float32),
        pltpu.VMEM((EPT,), jnp.int32),
        pltpu.VMEM((EPT,), jnp.int32),
        pltpu.VMEM((EPT,), jnp.float32),
        pltpu.VMEM((NP,), jnp.float32),
        pltpu.VMEM((SL,), jnp.float32),
        pltpu.VMEM((NS, SL), jnp.float32),
        pltpu.VMEM_SHARED((NS, NP), jnp.float32),
        pltpu.SemaphoreType.DMA,
    ],
)(_phase_a_body)


def _phase_b_body(h_hbm, src_hbm, dst_hbm, ex_hbm, out_hbm,
                  srcc, dstc, exc, gidx, sidx, rows, acc_sh,
                  msem, gsem, ssem):
    cid = lax.axis_index("c")
    sid = lax.axis_index("s")
    wid = cid * NS + sid
    tbase = wid * EPT

    def zrow(r, _):
        ridx = jnp.full((L,), r, jnp.int32)
        for c in range(D // L):
            cidx = c * L + lax.iota(jnp.int32, L)
            plsc.store_scatter(rows.at[0], [ridx, cidx],
                               jnp.zeros((L,), jnp.float32))
        return _
    lax.fori_loop(0, K, zrow, None)
    for z in range(SL // K):
        pltpu.sync_copy(rows.at[0], acc_sh.at[pl.ds(sid * SL + z * K, K)])
    plsc.subcore_barrier()

    def stage_meta(j, slot):
        base = tbase + j * K
        pltpu.async_copy(src_hbm.at[pl.ds(base, K)], srcc.at[slot],
                         msem.at[slot])
        pltpu.async_copy(dst_hbm.at[pl.ds(base, K)], dstc.at[slot],
                         msem.at[slot])
        pltpu.async_copy(ex_hbm.at[pl.ds(base, K)], exc.at[slot],
                         msem.at[slot])

    def wait_meta(slot):
        pltpu.make_async_copy(src_hbm.at[pl.ds(0, K)], srcc.at[slot],
                              msem.at[slot]).wait()
        pltpu.make_async_copy(dst_hbm.at[pl.ds(0, K)], dstc.at[slot],
                              msem.at[slot]).wait()
        pltpu.make_async_copy(ex_hbm.at[pl.ds(0, K)], exc.at[slot],
                              msem.at[slot]).wait()

    def copy_idx(src2d, mslot, dst2d, rslot):
        for u in range(K // L):
            dst2d[rslot, pl.ds(u * L, L)] = src2d[mslot, pl.ds(u * L, L)]

    def start_gather(rslot):
        pltpu.async_copy(h_hbm.at[gidx.at[rslot]], rows.at[rslot],
                         gsem.at[rslot])

    def wait_gather(rslot):
        pltpu.make_async_copy(h_hbm.at[gidx.at[rslot]], rows.at[rslot],
                              gsem.at[rslot]).wait()

    def start_scatter(rslot):
        pltpu.async_copy(rows.at[rslot], acc_sh.at[sidx.at[rslot]],
                         ssem.at[rslot], add=True)

    def wait_scatter(rslot):
        pltpu.make_async_copy(rows.at[rslot], acc_sh.at[sidx.at[rslot]],
                              ssem.at[rslot]).wait()

    def scale(mslot, rslot):
        def body(r4, _):
            for k in range(4):
                ridx = jnp.full((L,), r4 * 4 + k, jnp.int32)
                wb = plsc.load_gather(exc.at[mslot], [ridx])
                for c in range(D // L):
                    cidx = c * L + lax.iota(jnp.int32, L)
                    v = plsc.load_gather(rows.at[rslot], [ridx, cidx])
                    plsc.store_scatter(rows.at[rslot], [ridx, cidx], v * wb)
            return _
        lax.fori_loop(0, K // 4, body, None)

    def section(s, b, first):
        j = 8 * s + b
        rs, rn = b % 2, (b + 1) % 2
        mn = (b + 1) % 8
        wait_gather(rs)
        wb_slot = b
        scale(wb_slot, rs)
        copy_idx(dstc, wb_slot, sidx, rs)
        start_scatter(rs)
        if not first:
            wait_scatter(rn)
        wait_meta(mn)
        copy_idx(srcc, mn, gidx, rn)
        start_gather(rn)
        stage_meta(jnp.minimum(j + 8, NCHUNK - 1), wb_slot)

    for b in range(8):
        stage_meta(b, b)
    wait_meta(0)
    copy_idx(srcc, 0, gidx, 0)
    start_gather(0)

    for b in range(8):
        section(0, b, first=(b == 0))

    def superstep(s, _):
        for b in range(8):
            section(s, b, first=False)
        return _
    lax.fori_loop(1, NSS, superstep, None)

    wait_gather(0)
    wait_scatter(1)
    for b in range(1, 8):
        wait_meta(b)

    plsc.subcore_barrier()
    pltpu.sync_copy(acc_sh.at[pl.ds(sid * SL, SL)],
                    out_hbm.at[cid, pl.ds(sid * SL, SL)])


_phase_b = functools.partial(
    pl.kernel,
    out_type=jax.ShapeDtypeStruct((NC, NP, D), jnp.float32),
    mesh=plsc.VectorSubcoreMesh(core_axis_name="c", subcore_axis_name="s"),
    compiler_params=_SC_PARAMS,
    scratch_types=[
        pltpu.VMEM((8, K), jnp.int32),
        pltpu.VMEM((8, K), jnp.int32),
        pltpu.VMEM((8, K), jnp.float32),
        pltpu.VMEM((2, K), jnp.int32),
        pltpu.VMEM((2, K), jnp.int32),
        pltpu.VMEM((2, K, D), jnp.float32),
        pltpu.VMEM_SHARED((NP, D), jnp.float32),
        pltpu.SemaphoreType.DMA((8,)),
        pltpu.SemaphoreType.DMA((2,)),
        pltpu.SemaphoreType.DMA((2,)),
    ],
)(_phase_b_body)


_RB = 512


def _dense1_body(x_ref, w_ref, asr_ref, adr_ref, h_ref, as_ref, ad_ref):
    h = jnp.dot(x_ref[...], w_ref[...], preferred_element_type=jnp.float32)
    h_ref[...] = h
    as_ref[...] = jnp.sum(h * asr_ref[...], axis=1, keepdims=True)
    ad_ref[...] = jnp.sum(h * adr_ref[...], axis=1, keepdims=True)


def _dense2_body(p_ref, den_ref, b_ref, w_ref, asr_ref, adr_ref,
                 h_ref, as_ref, ad_ref):
    i = pl.program_id(0)
    rows = i * _RB + lax.broadcasted_iota(jnp.int32, (_RB, D), 0)
    den = den_ref[0] + den_ref[1] + 1e-16
    x = (p_ref[0] + p_ref[1]) / den + b_ref[...]
    x = jnp.where(rows < N, x, 0.0)
    h = jnp.dot(x, w_ref[...], preferred_element_type=jnp.float32)
    h_ref[...] = h
    as_ref[...] = jnp.sum(h * asr_ref[...], axis=1, keepdims=True)
    ad_ref[...] = jnp.sum(h * adr_ref[...], axis=1, keepdims=True)


def _dense1(x, w, a_src, a_dst):
    return pl.pallas_call(
        _dense1_body,
        grid=(NP // _RB,),
        in_specs=[
            pl.BlockSpec((_RB, D), lambda i: (i, 0)),
            pl.BlockSpec((D, D), lambda i: (0, 0)),
            pl.BlockSpec((1, D), lambda i: (0, 0)),
            pl.BlockSpec((1, D), lambda i: (0, 0)),
        ],
        out_specs=[
            pl.BlockSpec((_RB, D), lambda i: (i, 0)),
            pl.BlockSpec((_RB, 1), lambda i: (i, 0)),
            pl.BlockSpec((_RB, 1), lambda i: (i, 0)),
        ],
        out_shape=[
            jax.ShapeDtypeStruct((NP, D), jnp.float32),
            jax.ShapeDtypeStruct((NP, 1), jnp.float32),
            jax.ShapeDtypeStruct((NP, 1), jnp.float32),
        ],
    )(x, w, a_src.reshape(1, D), a_dst.reshape(1, D))


def _dense2(p, den, b, w, a_src, a_dst):
    return pl.pallas_call(
        _dense2_body,
        grid=(NP // _RB,),
        in_specs=[
            pl.BlockSpec((NC, _RB, D), lambda i: (0, i, 0)),
            pl.BlockSpec((NC, _RB, 1), lambda i: (0, i, 0)),
            pl.BlockSpec((1, D), lambda i: (0, 0)),
            pl.BlockSpec((D, D), lambda i: (0, 0)),
            pl.BlockSpec((1, D), lambda i: (0, 0)),
            pl.BlockSpec((1, D), lambda i: (0, 0)),
        ],
        out_specs=[
            pl.BlockSpec((_RB, D), lambda i: (i, 0)),
            pl.BlockSpec((_RB, 1), lambda i: (i, 0)),
            pl.BlockSpec((_RB, 1), lambda i: (i, 0)),
        ],
        out_shape=[
            jax.ShapeDtypeStruct((NP, D), jnp.float32),
            jax.ShapeDtypeStruct((NP, 1), jnp.float32),
            jax.ShapeDtypeStruct((NP, 1), jnp.float32),
        ],
    )(p, den.reshape(NC, NP, 1), b.reshape(1, D), w,
      a_src.reshape(1, D), a_dst.reshape(1, D))


_PB = 400


def _pool_body(p_ref, den_ref, b_ref, batch_ref, out_ref, acc, cnt):
    i = pl.program_id(0)
    den = den_ref[0] + den_ref[1] + 1e-16
    x = (p_ref[0] + p_ref[1]) / den + b_ref[...]
    onehot = (batch_ref[...] ==
              lax.broadcasted_iota(jnp.int32, (_PB, NG), 1)).astype(jnp.float32)
    psum = lax.dot_general(onehot, x, (((0,), (0,)), ((), ())),
                           preferred_element_type=jnp.float32)
    pcnt = lax.dot_general(onehot, jnp.ones((_PB, 1), jnp.float32),
                           (((0,), (0,)), ((), ())),
                           preferred_element_type=jnp.float32)

    @pl.when(i == 0)
    def _():
        acc[...] = jnp.zeros_like(acc)
        cnt[...] = jnp.zeros_like(cnt)

    acc[...] += psum
    cnt[...] += pcnt

    @pl.when(i == N // _PB - 1)
    def _():
        out_ref[...] = acc[...] / jnp.maximum(cnt[...], 1.0)


def _pool(p, den, b, batch):
    return pl.pallas_call(
        _pool_body,
        grid=(N // _PB,),
        in_specs=[
            pl.BlockSpec((NC, _PB, D), lambda i: (0, i, 0)),
            pl.BlockSpec((NC, _PB, 1), lambda i: (0, i, 0)),
            pl.BlockSpec((1, D), lambda i: (0, 0)),
            pl.BlockSpec((_PB, 1), lambda i: (i, 0)),
        ],
        out_specs=pl.BlockSpec((NG, D), lambda i: (0, 0)),
        out_shape=jax.ShapeDtypeStruct((NG, D), jnp.float32),
        scratch_shapes=[
            pltpu.VMEM((NG, D), jnp.float32),
            pltpu.VMEM((NG, 1), jnp.float32),
        ],
    )(p, den.reshape(NC, NP, 1), b.reshape(1, D), batch.reshape(N, 1))


def kernel(x, edge_index, batch,
           W1, a_src1, a_dst1, b1, W2, a_src2, a_dst2, b2,
           W3, a_src3, a_dst3, b3):
    loop = jnp.arange(N, dtype=jnp.int32)
    padi = N + jnp.arange(E2P - E - N, dtype=jnp.int32) % (NP - N)
    src = jnp.concatenate([edge_index[0], loop, padi])
    dst = jnp.concatenate([edge_index[1], loop, padi])
    xp = jnp.pad(x, ((0, NP - N), (0, 0)))

    h, asv, adv = _dense1(xp, W1, a_src1, a_dst1)
    for (w, a_s, a_d, b) in ((W2, a_src2, a_dst2, b1),
                             (W3, a_src3, a_dst3, b2)):
        den, ex = _phase_a(src, dst, asv.reshape(NP), adv.reshape(NP))
        p = _phase_b(h, src, dst, ex)
        h, asv, adv = _dense2(p, den, b, w, a_s, a_d)
    den, ex = _phase_a(src, dst, asv.reshape(NP), adv.reshape(NP))
    p = _phase_b(h, src, dst, ex)
    return _pool(p, den, b3, batch)


# speedup vs baseline: 4.2938x; 2.0710x over previous
import functools

import jax
import jax.numpy as jnp
from jax import lax
from jax.experimental import pallas as pl
from jax.experimental.pallas import tpu as pltpu
from jax.experimental.pallas import tpu_sc as plsc

N = 10000
E = 320000
D = 128
NG = 128
NEG = 0.2

NC, NS, L = 2, 16, 16
NW = NC * NS
NP = 10240
K = 128
NCHUNK = 88
EPT = NCHUNK * K
E2P = NW * EPT
SL = NP // NS
NSS = NCHUNK // 8


def _leaky(v):
    return jnp.maximum(v, NEG * v)


def _gmax_of(as_v):
    def body(i, acc):
        return jnp.maximum(acc, as_v[pl.ds(i * L, L)])
    m = lax.fori_loop(0, NP // L, body, jnp.full((L,), -jnp.inf, jnp.float32))
    return jnp.max(m)


def _phase_a_body(src_hbm, dst_hbm, as_hbm, ad_hbm, den_hbm, ex_hbm,
                  as_v, ad_v, srcv, dstv, exv, dloc, accv, tmp2, dsh, sem):
    cid = lax.axis_index("c")
    sid = lax.axis_index("s")
    wid = cid * NS + sid
    pltpu.sync_copy(as_hbm, as_v)
    pltpu.sync_copy(ad_hbm, ad_v)
    pltpu.sync_copy(src_hbm.at[pl.ds(wid * EPT, EPT)], srcv)
    pltpu.sync_copy(dst_hbm.at[pl.ds(wid * EPT, EPT)], dstv)

    def zero(i, _):
        dloc[pl.ds(i * L, L)] = jnp.zeros((L,), jnp.float32)
        return _
    lax.fori_loop(0, NP // L, zero, None)

    gmax = _gmax_of(as_v)

    def edge(i, _):
        s = srcv[pl.ds(i * L, L)]
        d = dstv[pl.ds(i * L, L)]
        a1 = plsc.load_gather(as_v, [s])
        a2 = plsc.load_gather(ad_v, [d])
        e = _leaky(a1 + a2)
        m = _leaky(gmax + a2)
        ex = jnp.exp(e - m)
        exv[pl.ds(i * L, L)] = ex
        plsc.addupdate_scatter(dloc, [d], ex)
        return _
    lax.fori_loop(0, EPT // L, edge, None)
    pltpu.sync_copy(exv, ex_hbm.at[pl.ds(wid * EPT, EPT)])

    pltpu.sync_copy(dloc, dsh.at[sid])
    plsc.subcore_barrier()
    pltpu.sync_copy(dsh.at[:, pl.ds(sid * SL, SL)], tmp2)

    def macc(i, _):
        tot = tmp2[0, pl.ds(i * L, L)]
        for t in range(1, NS):
            tot = tot + tmp2[t, pl.ds(i * L, L)]
        accv[pl.ds(i * L, L)] = tot
        return _
    lax.fori_loop(0, SL // L, macc, None)
    pltpu.sync_copy(accv, den_hbm.at[cid, pl.ds(sid * SL, SL)])


_SC_PARAMS = pltpu.CompilerParams(needs_layout_passes=False)

_phase_a = functools.partial(
    pl.kernel,
    out_type=(jax.ShapeDtypeStruct((NC, NP), jnp.float32),
              jax.ShapeDtypeStruct((E2P,), jnp.float32)),
    mesh=plsc.VectorSubcoreMesh(core_axis_name="c", subcore_axis_name="s"),
    compiler_params=_SC_PARAMS,
    scratch_types=[
        pltpu.VMEM((NP,), jnp.float32),
        pltpu.VMEM((NP,), jnp.float32),
        pltpu.VMEM((EPT,), jnp.int32),
        pltpu.VMEM((EPT,), jnp.int32),
        pltpu.VMEM((EPT,), jnp.float32),
        pltpu.VMEM((NP,), jnp.float32),
        pltpu.VMEM((SL,), jnp.float32),
        pltpu.VMEM((NS, SL), jnp.float32),
        pltpu.VMEM_SHARED((NS, NP), jnp.float32),
        pltpu.SemaphoreType.DMA,
    ],
)(_phase_a_body)


def _phase_b_body(h_hbm, src_hbm, dst_hbm, ex_hbm, out_hbm,
                  srcc, dstc, exc, gidx, sidx, rows, acc_sh,
                  msem, gsem, ssem):
    cid = lax.axis_index("c")
    sid = lax.axis_index("s")
    wid = cid * NS + sid
    tbase = wid * EPT

    def zrow(r, _):
        ridx = jnp.full((L,), r, jnp.int32)
        for c in range(D // L):
            cidx = c * L + lax.iota(jnp.int32, L)
            plsc.store_scatter(rows.at[0], [ridx, cidx],
                               jnp.zeros((L,), jnp.float32))
        return _
    lax.fori_loop(0, K, zrow, None)
    for z in range(SL // K):
        pltpu.sync_copy(rows.at[0], acc_sh.at[pl.ds(sid * SL + z * K, K)])
    plsc.subcore_barrier()

    def stage_meta(j, slot):
        base = tbase + j * K
        pltpu.async_copy(src_hbm.at[pl.ds(base, K)], srcc.at[slot],
                         msem.at[slot])
        pltpu.async_copy(dst_hbm.at[pl.ds(base, K)], dstc.at[slot],
                         msem.at[slot])
        pltpu.async_copy(ex_hbm.at[pl.ds(base, K)], exc.at[slot],
                         msem.at[slot])

    def wait_meta(slot):
        pltpu.make_async_copy(src_hbm.at[pl.ds(0, K)], srcc.at[slot],
                              msem.at[slot]).wait()
        pltpu.make_async_copy(dst_hbm.at[pl.ds(0, K)], dstc.at[slot],
                              msem.at[slot]).wait()
        pltpu.make_async_copy(ex_hbm.at[pl.ds(0, K)], exc.at[slot],
                              msem.at[slot]).wait()

    def copy_idx(src2d, mslot, dst2d, rslot):
        for u in range(K // L):
            dst2d[rslot, pl.ds(u * L, L)] = src2d[mslot, pl.ds(u * L, L)]

    def start_gather(rslot):
        pltpu.async_copy(h_hbm.at[gidx.at[rslot]], rows.at[rslot],
                         gsem.at[rslot])

    def wait_gather(rslot):
        pltpu.make_async_copy(h_hbm.at[gidx.at[rslot]], rows.at[rslot],
                              gsem.at[rslot]).wait()

    def start_scatter(rslot):
        pltpu.async_copy(rows.at[rslot], acc_sh.at[sidx.at[rslot]],
                         ssem.at[rslot], add=True)

    def wait_scatter(rslot):
        pltpu.make_async_copy(rows.at[rslot], acc_sh.at[sidx.at[rslot]],
                              ssem.at[rslot]).wait()

    def scale(mslot, rslot):
        @plsc.parallel_loop(0, K // 4, unroll=2)
        def _(r4):
            for k in range(4):
                ridx = jnp.full((L,), r4 * 4 + k, jnp.int32)
                wb = plsc.load_gather(exc.at[mslot], [ridx])
                for c in range(D // L):
                    cidx = c * L + lax.iota(jnp.int32, L)
                    v = plsc.load_gather(rows.at[rslot], [ridx, cidx])
                    plsc.store_scatter(rows.at[rslot], [ridx, cidx], v * wb)

    def section(s, b):
        j = 8 * s + b
        rs, rn = b % 2, (b + 1) % 2
        mn = (b + 1) % 8
        wait_gather(rs)
        wb_slot = b
        scale(wb_slot, rs)
        copy_idx(dstc, wb_slot, sidx, rs)
        start_scatter(rs)
        if b == 0:
            @pl.when(s > 0)
            def _():
                wait_scatter(rn)
        else:
            wait_scatter(rn)
        wait_meta(mn)
        copy_idx(srcc, mn, gidx, rn)
        start_gather(rn)
        stage_meta(jnp.minimum(j + 8, NCHUNK - 1), wb_slot)

    for b in range(8):
        stage_meta(b, b)
    wait_meta(0)
    copy_idx(srcc, 0, gidx, 0)
    start_gather(0)

    def superstep(s, _):
        for b in range(8):
            section(s, b)
        return _
    lax.fori_loop(0, NSS, superstep, None)

    wait_gather(0)
    wait_scatter(1)
    for b in range(1, 8):
        wait_meta(b)

    plsc.subcore_barrier()
    pltpu.sync_copy(acc_sh.at[pl.ds(sid * SL, SL)],
                    out_hbm.at[cid, pl.ds(sid * SL, SL)])


_phase_b = functools.partial(
    pl.kernel,
    out_type=jax.ShapeDtypeStruct((NC, NP, D), jnp.float32),
    mesh=plsc.VectorSubcoreMesh(core_axis_name="c", subcore_axis_name="s"),
    compiler_params=_SC_PARAMS,
    scratch_types=[
        pltpu.VMEM((8, K), jnp.int32),
        pltpu.VMEM((8, K), jnp.int32),
        pltpu.VMEM((8, K), jnp.float32),
        pltpu.VMEM((2, K), jnp.int32),
        pltpu.VMEM((2, K), jnp.int32),
        pltpu.VMEM((2, K, D), jnp.float32),
        pltpu.VMEM_SHARED((NP, D), jnp.float32),
        pltpu.SemaphoreType.DMA((8,)),
        pltpu.SemaphoreType.DMA((2,)),
        pltpu.SemaphoreType.DMA((2,)),
    ],
)(_phase_b_body)


_RB = 512


def _dense1_body(x_ref, w_ref, asr_ref, adr_ref, h_ref, as_ref, ad_ref):
    h = jnp.dot(x_ref[...], w_ref[...], preferred_element_type=jnp.float32)
    h_ref[...] = h
    as_ref[...] = jnp.sum(h * asr_ref[...], axis=1, keepdims=True)
    ad_ref[...] = jnp.sum(h * adr_ref[...], axis=1, keepdims=True)


def _dense2_body(p_ref, den_ref, b_ref, w_ref, asr_ref, adr_ref,
                 h_ref, as_ref, ad_ref):
    i = pl.program_id(0)
    rows = i * _RB + lax.broadcasted_iota(jnp.int32, (_RB, D), 0)
    den = den_ref[0] + den_ref[1] + 1e-16
    x = (p_ref[0] + p_ref[1]) / den + b_ref[...]
    x = jnp.where(rows < N, x, 0.0)
    h = jnp.dot(x, w_ref[...], preferred_element_type=jnp.float32)
    h_ref[...] = h
    as_ref[...] = jnp.sum(h * asr_ref[...], axis=1, keepdims=True)
    ad_ref[...] = jnp.sum(h * adr_ref[...], axis=1, keepdims=True)


def _dense1(x, w, a_src, a_dst):
    return pl.pallas_call(
        _dense1_body,
        grid=(NP // _RB,),
        in_specs=[
            pl.BlockSpec((_RB, D), lambda i: (i, 0)),
            pl.BlockSpec((D, D), lambda i: (0, 0)),
            pl.BlockSpec((1, D), lambda i: (0, 0)),
            pl.BlockSpec((1, D), lambda i: (0, 0)),
        ],
        out_specs=[
            pl.BlockSpec((_RB, D), lambda i: (i, 0)),
            pl.BlockSpec((_RB, 1), lambda i: (i, 0)),
            pl.BlockSpec((_RB, 1), lambda i: (i, 0)),
        ],
        out_shape=[
            jax.ShapeDtypeStruct((NP, D), jnp.float32),
            jax.ShapeDtypeStruct((NP, 1), jnp.float32),
            jax.ShapeDtypeStruct((NP, 1), jnp.float32),
        ],
    )(x, w, a_src.reshape(1, D), a_dst.reshape(1, D))


def _dense2(p, den, b, w, a_src, a_dst):
    return pl.pallas_call(
        _dense2_body,
        grid=(NP // _RB,),
        in_specs=[
            pl.BlockSpec((NC, _RB, D), lambda i: (0, i, 0)),
            pl.BlockSpec((NC, _RB, 1), lambda i: (0, i, 0)),
            pl.BlockSpec((1, D), lambda i: (0, 0)),
            pl.BlockSpec((D, D), lambda i: (0, 0)),
            pl.BlockSpec((1, D), lambda i: (0, 0)),
            pl.BlockSpec((1, D), lambda i: (0, 0)),
        ],
        out_specs=[
            pl.BlockSpec((_RB, D), lambda i: (i, 0)),
            pl.BlockSpec((_RB, 1), lambda i: (i, 0)),
            pl.BlockSpec((_RB, 1), lambda i: (i, 0)),
        ],
        out_shape=[
            jax.ShapeDtypeStruct((NP, D), jnp.float32),
            jax.ShapeDtypeStruct((NP, 1), jnp.float32),
            jax.ShapeDtypeStruct((NP, 1), jnp.float32),
        ],
    )(p, den.reshape(NC, NP, 1), b.reshape(1, D), w,
      a_src.reshape(1, D), a_dst.reshape(1, D))


_PB = 400


def _pool_body(p_ref, den_ref, b_ref, batch_ref, out_ref, acc, cnt):
    i = pl.program_id(0)
    den = den_ref[0] + den_ref[1] + 1e-16
    x = (p_ref[0] + p_ref[1]) / den + b_ref[...]
    onehot = (batch_ref[...] ==
              lax.broadcasted_iota(jnp.int32, (_PB, NG), 1)).astype(jnp.float32)
    psum = lax.dot_general(onehot, x, (((0,), (0,)), ((), ())),
                           preferred_element_type=jnp.float32)
    pcnt = lax.dot_general(onehot, jnp.ones((_PB, 1), jnp.float32),
                           (((0,), (0,)), ((), ())),
                           preferred_element_type=jnp.float32)

    @pl.when(i == 0)
    def _():
        acc[...] = jnp.zeros_like(acc)
        cnt[...] = jnp.zeros_like(cnt)

    acc[...] += psum
    cnt[...] += pcnt

    @pl.when(i == N // _PB - 1)
    def _():
        out_ref[...] = acc[...] / jnp.maximum(cnt[...], 1.0)


def _pool(p, den, b, batch):
    return pl.pallas_call(
        _pool_body,
        grid=(N // _PB,),
        in_specs=[
            pl.BlockSpec((NC, _PB, D), lambda i: (0, i, 0)),
            pl.BlockSpec((NC, _PB, 1), lambda i: (0, i, 0)),
            pl.BlockSpec((1, D), lambda i: (0, 0)),
            pl.BlockSpec((_PB, 1), lambda i: (i, 0)),
        ],
        out_specs=pl.BlockSpec((NG, D), lambda i: (0, 0)),
        out_shape=jax.ShapeDtypeStruct((NG, D), jnp.float32),
        scratch_shapes=[
            pltpu.VMEM((NG, D), jnp.float32),
            pltpu.VMEM((NG, 1), jnp.float32),
        ],
    )(p, den.reshape(NC, NP, 1), b.reshape(1, D), batch.reshape(N, 1))


def kernel(x, edge_index, batch,
           W1, a_src1, a_dst1, b1, W2, a_src2, a_dst2, b2,
           W3, a_src3, a_dst3, b3):
    loop = jnp.arange(N, dtype=jnp.int32)
    padi = N + jnp.arange(E2P - E - N, dtype=jnp.int32) % (NP - N)
    src = jnp.concatenate([edge_index[0], loop, padi])
    dst = jnp.concatenate([edge_index[1], loop, padi])
    xp = jnp.pad(x, ((0, NP - N), (0, 0)))

    h, asv, adv = _dense1(xp, W1, a_src1, a_dst1)
    for (w, a_s, a_d, b) in ((W2, a_src2, a_dst2, b1),
                             (W3, a_src3, a_dst3, b2)):
        den, ex = _phase_a(src, dst, asv.reshape(NP), adv.reshape(NP))
        p = _phase_b(h, src, dst, ex)
        h, asv, adv = _dense2(p, den, b, w, a_s, a_d)
    den, ex = _phase_a(src, dst, asv.reshape(NP), adv.reshape(NP))
    p = _phase_b(h, src, dst, ex)
    return _pool(p, den, b3, batch)


# speedup vs baseline: 4.9272x; 1.1475x over previous
import functools

import jax
import jax.numpy as jnp
from jax import lax
from jax.experimental import pallas as pl
from jax.experimental.pallas import tpu as pltpu
from jax.experimental.pallas import tpu_sc as plsc

N = 10000
E = 320000
D = 128
NG = 128
NEG = 0.2

NC, NS, L = 2, 16, 16
NW = NC * NS
NP = 10240
K = 128
NCHUNK = 88
EPT = NCHUNK * K
E2P = NW * EPT
SL = NP // NS
NSS = NCHUNK // 8


def _leaky(v):
    return jnp.maximum(v, NEG * v)


def _gmax_of(as_v):
    def body(i, acc):
        return jnp.maximum(acc, as_v[pl.ds(i * L, L)])
    m = lax.fori_loop(0, NP // L, body, jnp.full((L,), -jnp.inf, jnp.float32))
    return jnp.max(m)


def _phase_a_body(src_hbm, dst_hbm, as_hbm, ad_hbm, den_hbm, ex_hbm,
                  as_v, ad_v, srcv, dstv, exv, dloc, accv, tmp2, dsh, sem):
    cid = lax.axis_index("c")
    sid = lax.axis_index("s")
    wid = cid * NS + sid
    pltpu.sync_copy(as_hbm, as_v)
    pltpu.sync_copy(ad_hbm, ad_v)
    pltpu.sync_copy(src_hbm.at[pl.ds(wid * EPT, EPT)], srcv)
    pltpu.sync_copy(dst_hbm.at[pl.ds(wid * EPT, EPT)], dstv)

    def zero(i, _):
        dloc[pl.ds(i * L, L)] = jnp.zeros((L,), jnp.float32)
        return _
    lax.fori_loop(0, NP // L, zero, None)

    gmax = _gmax_of(as_v)

    def edge(i, _):
        s = srcv[pl.ds(i * L, L)]
        d = dstv[pl.ds(i * L, L)]
        a1 = plsc.load_gather(as_v, [s])
        a2 = plsc.load_gather(ad_v, [d])
        e = _leaky(a1 + a2)
        m = _leaky(gmax + a2)
        ex = jnp.exp(e - m)
        exv[pl.ds(i * L, L)] = ex
        plsc.addupdate_scatter(dloc, [d], ex)
        return _
    lax.fori_loop(0, EPT // L, edge, None)
    pltpu.sync_copy(exv, ex_hbm.at[pl.ds(wid * EPT, EPT)])

    pltpu.sync_copy(dloc, dsh.at[sid])
    plsc.subcore_barrier()
    pltpu.sync_copy(dsh.at[:, pl.ds(sid * SL, SL)], tmp2)

    def macc(i, _):
        tot = tmp2[0, pl.ds(i * L, L)]
        for t in range(1, NS):
            tot = tot + tmp2[t, pl.ds(i * L, L)]
        accv[pl.ds(i * L, L)] = tot
        return _
    lax.fori_loop(0, SL // L, macc, None)
    pltpu.sync_copy(accv, den_hbm.at[cid, pl.ds(sid * SL, SL)])


_SC_PARAMS = pltpu.CompilerParams(needs_layout_passes=False)

_phase_a = functools.partial(
    pl.kernel,
    out_type=(jax.ShapeDtypeStruct((NC, NP), jnp.float32),
              jax.ShapeDtypeStruct((E2P,), jnp.float32)),
    mesh=plsc.VectorSubcoreMesh(core_axis_name="c", subcore_axis_name="s"),
    compiler_params=_SC_PARAMS,
    scratch_types=[
        pltpu.VMEM((NP,), jnp.float32),
        pltpu.VMEM((NP,), jnp.float32),
        pltpu.VMEM((EPT,), jnp.int32),
        pltpu.VMEM((EPT,), jnp.int32),
        pltpu.VMEM((EPT,), jnp.float32),
        pltpu.VMEM((NP,), jnp.float32),
        pltpu.VMEM((SL,), jnp.float32),
        pltpu.VMEM((NS, SL), jnp.float32),
        pltpu.VMEM_SHARED((NS, NP), jnp.float32),
        pltpu.SemaphoreType.DMA,
    ],
)(_phase_a_body)


def _phase_b_body(h_hbm, src_hbm, dst_hbm, ex_hbm, out_hbm,
                  srcc, dstc, exc, gidx, sidx, rows, acc_sh,
                  msem, gsem, ssem):
    cid = lax.axis_index("c")
    sid = lax.axis_index("s")
    wid = cid * NS + sid
    tbase = wid * EPT

    def zrow(r, _):
        ridx = jnp.full((L,), r, jnp.int32)
        for c in range(D // L):
            cidx = c * L + lax.iota(jnp.int32, L)
            plsc.store_scatter(rows.at[0], [ridx, cidx],
                               jnp.zeros((L,), jnp.float32))
        return _
    lax.fori_loop(0, K, zrow, None)
    for z in range(SL // K):
        pltpu.sync_copy(rows.at[0], acc_sh.at[pl.ds(sid * SL + z * K, K)])
    plsc.subcore_barrier()

    def stage_meta(j, slot):
        base = tbase + j * K
        pltpu.async_copy(src_hbm.at[pl.ds(base, K)], srcc.at[slot],
                         msem.at[slot])
        pltpu.async_copy(dst_hbm.at[pl.ds(base, K)], dstc.at[slot],
                         msem.at[slot])
        pltpu.async_copy(ex_hbm.at[pl.ds(base, K)], exc.at[slot],
                         msem.at[slot])

    def wait_meta(slot):
        pltpu.make_async_copy(src_hbm.at[pl.ds(0, K)], srcc.at[slot],
                              msem.at[slot]).wait()
        pltpu.make_async_copy(dst_hbm.at[pl.ds(0, K)], dstc.at[slot],
                              msem.at[slot]).wait()
        pltpu.make_async_copy(ex_hbm.at[pl.ds(0, K)], exc.at[slot],
                              msem.at[slot]).wait()

    def copy_idx(src2d, mslot, dst2d, rslot):
        for u in range(K // L):
            dst2d[rslot, pl.ds(u * L, L)] = src2d[mslot, pl.ds(u * L, L)]

    def start_gather(rslot):
        pltpu.async_copy(h_hbm.at[gidx.at[rslot]], rows.at[rslot],
                         gsem.at[rslot])

    def wait_gather(rslot):
        pltpu.make_async_copy(h_hbm.at[gidx.at[rslot]], rows.at[rslot],
                              gsem.at[rslot]).wait()

    def start_scatter(rslot):
        pltpu.async_copy(rows.at[rslot], acc_sh.at[sidx.at[rslot]],
                         ssem.at[rslot], add=True)

    def wait_scatter(rslot):
        pltpu.make_async_copy(rows.at[rslot], acc_sh.at[sidx.at[rslot]],
                              ssem.at[rslot]).wait()

    def scale(mslot, rslot):
        @plsc.parallel_loop(0, K // 4, unroll=2)
        def _(r4):
            for k in range(4):
                ridx = jnp.full((L,), r4 * 4 + k, jnp.int32)
                wb = plsc.load_gather(exc.at[mslot], [ridx])
                for c in range(D // L):
                    cidx = c * L + lax.iota(jnp.int32, L)
                    v = plsc.load_gather(rows.at[rslot], [ridx, cidx])
                    plsc.store_scatter(rows.at[rslot], [ridx, cidx], v * wb)

    def section(s, b):
        j = 8 * s + b
        rs, rn = b % 2, (b + 1) % 2
        mn = (b + 1) % 8
        wb_slot = b
        wait_gather(rs)
        if b == 0:
            @pl.when(s > 0)
            def _():
                wait_scatter(rn)
        else:
            wait_scatter(rn)
        wait_meta(mn)
        copy_idx(srcc, mn, gidx, rn)
        start_gather(rn)
        scale(wb_slot, rs)
        copy_idx(dstc, wb_slot, sidx, rs)
        start_scatter(rs)
        stage_meta(jnp.minimum(j + 8, NCHUNK - 1), wb_slot)

    for b in range(8):
        stage_meta(b, b)
    wait_meta(0)
    copy_idx(srcc, 0, gidx, 0)
    start_gather(0)

    def superstep(s, _):
        for b in range(8):
            section(s, b)
        return _
    lax.fori_loop(0, NSS, superstep, None)

    wait_gather(0)
    wait_scatter(1)
    for b in range(1, 8):
        wait_meta(b)

    plsc.subcore_barrier()
    pltpu.sync_copy(acc_sh.at[pl.ds(sid * SL, SL)],
                    out_hbm.at[cid, pl.ds(sid * SL, SL)])


_phase_b = functools.partial(
    pl.kernel,
    out_type=jax.ShapeDtypeStruct((NC, NP, D), jnp.float32),
    mesh=plsc.VectorSubcoreMesh(core_axis_name="c", subcore_axis_name="s"),
    compiler_params=_SC_PARAMS,
    scratch_types=[
        pltpu.VMEM((8, K), jnp.int32),
        pltpu.VMEM((8, K), jnp.int32),
        pltpu.VMEM((8, K), jnp.float32),
        pltpu.VMEM((2, K), jnp.int32),
        pltpu.VMEM((2, K), jnp.int32),
        pltpu.VMEM((2, K, D), jnp.float32),
        pltpu.VMEM_SHARED((NP, D), jnp.float32),
        pltpu.SemaphoreType.DMA((8,)),
        pltpu.SemaphoreType.DMA((2,)),
        pltpu.SemaphoreType.DMA((2,)),
    ],
)(_phase_b_body)


_RB = 512


def _dense1_body(x_ref, w_ref, asr_ref, adr_ref, h_ref, as_ref, ad_ref):
    h = jnp.dot(x_ref[...], w_ref[...], preferred_element_type=jnp.float32)
    h_ref[...] = h
    as_ref[...] = jnp.sum(h * asr_ref[...], axis=1, keepdims=True)
    ad_ref[...] = jnp.sum(h * adr_ref[...], axis=1, keepdims=True)


def _dense2_body(p_ref, den_ref, b_ref, w_ref, asr_ref, adr_ref,
                 h_ref, as_ref, ad_ref):
    i = pl.program_id(0)
    rows = i * _RB + lax.broadcasted_iota(jnp.int32, (_RB, D), 0)
    den = den_ref[0] + den_ref[1] + 1e-16
    x = (p_ref[0] + p_ref[1]) / den + b_ref[...]
    x = jnp.where(rows < N, x, 0.0)
    h = jnp.dot(x, w_ref[...], preferred_element_type=jnp.float32)
    h_ref[...] = h
    as_ref[...] = jnp.sum(h * asr_ref[...], axis=1, keepdims=True)
    ad_ref[...] = jnp.sum(h * adr_ref[...], axis=1, keepdims=True)


def _dense1(x, w, a_src, a_dst):
    return pl.pallas_call(
        _dense1_body,
        grid=(NP // _RB,),
        in_specs=[
            pl.BlockSpec((_RB, D), lambda i: (i, 0)),
            pl.BlockSpec((D, D), lambda i: (0, 0)),
            pl.BlockSpec((1, D), lambda i: (0, 0)),
            pl.BlockSpec((1, D), lambda i: (0, 0)),
        ],
        out_specs=[
            pl.BlockSpec((_RB, D), lambda i: (i, 0)),
            pl.BlockSpec((_RB, 1), lambda i: (i, 0)),
            pl.BlockSpec((_RB, 1), lambda i: (i, 0)),
        ],
        out_shape=[
            jax.ShapeDtypeStruct((NP, D), jnp.float32),
            jax.ShapeDtypeStruct((NP, 1), jnp.float32),
            jax.ShapeDtypeStruct((NP, 1), jnp.float32),
        ],
    )(x, w, a_src.reshape(1, D), a_dst.reshape(1, D))


def _dense2(p, den, b, w, a_src, a_dst):
    return pl.pallas_call(
        _dense2_body,
        grid=(NP // _RB,),
        in_specs=[
            pl.BlockSpec((NC, _RB, D), lambda i: (0, i, 0)),
            pl.BlockSpec((NC, _RB, 1), lambda i: (0, i, 0)),
            pl.BlockSpec((1, D), lambda i: (0, 0)),
            pl.BlockSpec((D, D), lambda i: (0, 0)),
            pl.BlockSpec((1, D), lambda i: (0, 0)),
            pl.BlockSpec((1, D), lambda i: (0, 0)),
        ],
        out_specs=[
            pl.BlockSpec((_RB, D), lambda i: (i, 0)),
            pl.BlockSpec((_RB, 1), lambda i: (i, 0)),
            pl.BlockSpec((_RB, 1), lambda i: (i, 0)),
        ],
        out_shape=[
            jax.ShapeDtypeStruct((NP, D), jnp.float32),
            jax.ShapeDtypeStruct((NP, 1), jnp.float32),
            jax.ShapeDtypeStruct((NP, 1), jnp.float32),
        ],
    )(p, den.reshape(NC, NP, 1), b.reshape(1, D), w,
      a_src.reshape(1, D), a_dst.reshape(1, D))


_PB = 400


def _pool_body(p_ref, den_ref, b_ref, batch_ref, out_ref, acc, cnt):
    i = pl.program_id(0)
    den = den_ref[0] + den_ref[1] + 1e-16
    x = (p_ref[0] + p_ref[1]) / den + b_ref[...]
    onehot = (batch_ref[...] ==
              lax.broadcasted_iota(jnp.int32, (_PB, NG), 1)).astype(jnp.float32)
    psum = lax.dot_general(onehot, x, (((0,), (0,)), ((), ())),
                           preferred_element_type=jnp.float32)
    pcnt = lax.dot_general(onehot, jnp.ones((_PB, 1), jnp.float32),
                           (((0,), (0,)), ((), ())),
                           preferred_element_type=jnp.float32)

    @pl.when(i == 0)
    def _():
        acc[...] = jnp.zeros_like(acc)
        cnt[...] = jnp.zeros_like(cnt)

    acc[...] += psum
    cnt[...] += pcnt

    @pl.when(i == N // _PB - 1)
    def _():
        out_ref[...] = acc[...] / jnp.maximum(cnt[...], 1.0)


def _pool(p, den, b, batch):
    return pl.pallas_call(
        _pool_body,
        grid=(N // _PB,),
        in_specs=[
            pl.BlockSpec((NC, _PB, D), lambda i: (0, i, 0)),
            pl.BlockSpec((NC, _PB, 1), lambda i: (0, i, 0)),
            pl.BlockSpec((1, D), lambda i: (0, 0)),
            pl.BlockSpec((_PB, 1), lambda i: (i, 0)),
        ],
        out_specs=pl.BlockSpec((NG, D), lambda i: (0, 0)),
        out_shape=jax.ShapeDtypeStruct((NG, D), jnp.float32),
        scratch_shapes=[
            pltpu.VMEM((NG, D), jnp.float32),
            pltpu.VMEM((NG, 1), jnp.float32),
        ],
    )(p, den.reshape(NC, NP, 1), b.reshape(1, D), batch.reshape(N, 1))


def kernel(x, edge_index, batch,
           W1, a_src1, a_dst1, b1, W2, a_src2, a_dst2, b2,
           W3, a_src3, a_dst3, b3):
    loop = jnp.arange(N, dtype=jnp.int32)
    padi = N + jnp.arange(E2P - E - N, dtype=jnp.int32) % (NP - N)
    src = jnp.concatenate([edge_index[0], loop, padi])
    dst = jnp.concatenate([edge_index[1], loop, padi])
    xp = jnp.pad(x, ((0, NP - N), (0, 0)))

    h, asv, adv = _dense1(xp, W1, a_src1, a_dst1)
    for (w, a_s, a_d, b) in ((W2, a_src2, a_dst2, b1),
                             (W3, a_src3, a_dst3, b2)):
        den, ex = _phase_a(src, dst, asv.reshape(NP), adv.reshape(NP))
        p = _phase_b(h, src, dst, ex)
        h, asv, adv = _dense2(p, den, b, w, a_s, a_d)
    den, ex = _phase_a(src, dst, asv.reshape(NP), adv.reshape(NP))
    p = _phase_b(h, src, dst, ex)
    return _pool(p, den, b3, batch)


# speedup vs baseline: 5.1834x; 1.0520x over previous
import functools

import jax
import jax.numpy as jnp
from jax import lax
from jax.experimental import pallas as pl
from jax.experimental.pallas import tpu as pltpu
from jax.experimental.pallas import tpu_sc as plsc

N = 10000
E = 320000
D = 128
NG = 128
NEG = 0.2

NC, NS, L = 2, 16, 16
NW = NC * NS
NP = 10240
K = 96
NCHUNK = 108
EPT = NCHUNK * K
E2P = NW * EPT
SL = NP // NS
NSS = NCHUNK // 6


def _leaky(v):
    return jnp.maximum(v, NEG * v)


def _gmax_of(as_v):
    def body(i, acc):
        return jnp.maximum(acc, as_v[pl.ds(i * L, L)])
    m = lax.fori_loop(0, NP // L, body, jnp.full((L,), -jnp.inf, jnp.float32))
    return jnp.max(m)


def _phase_a_body(src_hbm, dst_hbm, as_hbm, ad_hbm, den_hbm, ex_hbm,
                  as_v, ad_v, srcv, dstv, exv, dloc, accv, tmp2, dsh, sem):
    cid = lax.axis_index("c")
    sid = lax.axis_index("s")
    wid = cid * NS + sid
    pltpu.sync_copy(as_hbm, as_v)
    pltpu.sync_copy(ad_hbm, ad_v)
    pltpu.sync_copy(src_hbm.at[pl.ds(wid * EPT, EPT)], srcv)
    pltpu.sync_copy(dst_hbm.at[pl.ds(wid * EPT, EPT)], dstv)

    def zero(i, _):
        dloc[pl.ds(i * L, L)] = jnp.zeros((L,), jnp.float32)
        return _
    lax.fori_loop(0, NP // L, zero, None)

    gmax = _gmax_of(as_v)

    def edge(i, _):
        s = srcv[pl.ds(i * L, L)]
        d = dstv[pl.ds(i * L, L)]
        a1 = plsc.load_gather(as_v, [s])
        a2 = plsc.load_gather(ad_v, [d])
        e = _leaky(a1 + a2)
        m = _leaky(gmax + a2)
        ex = jnp.exp(e - m)
        exv[pl.ds(i * L, L)] = ex
        plsc.addupdate_scatter(dloc, [d], ex)
        return _
    lax.fori_loop(0, EPT // L, edge, None)
    pltpu.sync_copy(exv, ex_hbm.at[pl.ds(wid * EPT, EPT)])

    pltpu.sync_copy(dloc, dsh.at[sid])
    plsc.subcore_barrier()
    pltpu.sync_copy(dsh.at[:, pl.ds(sid * SL, SL)], tmp2)

    def macc(i, _):
        tot = tmp2[0, pl.ds(i * L, L)]
        for t in range(1, NS):
            tot = tot + tmp2[t, pl.ds(i * L, L)]
        accv[pl.ds(i * L, L)] = tot
        return _
    lax.fori_loop(0, SL // L, macc, None)
    pltpu.sync_copy(accv, den_hbm.at[cid, pl.ds(sid * SL, SL)])


_SC_PARAMS = pltpu.CompilerParams(needs_layout_passes=False)

_phase_a = functools.partial(
    pl.kernel,
    out_type=(jax.ShapeDtypeStruct((NC, NP), jnp.float32),
              jax.ShapeDtypeStruct((E2P,), jnp.float32)),
    mesh=plsc.VectorSubcoreMesh(core_axis_name="c", subcore_axis_name="s"),
    compiler_params=_SC_PARAMS,
    scratch_types=[
        pltpu.VMEM((NP,), jnp.float32),
        pltpu.VMEM((NP,), jnp.float32),
        pltpu.VMEM((EPT,), jnp.int32),
        pltpu.VMEM((EPT,), jnp.int32),
        pltpu.VMEM((EPT,), jnp.float32),
        pltpu.VMEM((NP,), jnp.float32),
        pltpu.VMEM((SL,), jnp.float32),
        pltpu.VMEM((NS, SL), jnp.float32),
        pltpu.VMEM_SHARED((NS, NP), jnp.float32),
        pltpu.SemaphoreType.DMA,
    ],
)(_phase_a_body)


def _phase_b_body(h_hbm, src_hbm, dst_hbm, ex_hbm, out_hbm,
                  srcc, dstc, exc, gidx, sidx, rows, acc_sh,
                  msem, gsem, ssem):
    cid = lax.axis_index("c")
    sid = lax.axis_index("s")
    wid = cid * NS + sid
    tbase = wid * EPT

    def zrow(r, _):
        ridx = jnp.full((L,), r, jnp.int32)
        for c in range(D // L):
            cidx = c * L + lax.iota(jnp.int32, L)
            plsc.store_scatter(rows.at[0], [ridx, cidx],
                               jnp.zeros((L,), jnp.float32))
        return _
    lax.fori_loop(0, K, zrow, None)
    for z in range(SL // K):
        pltpu.sync_copy(rows.at[0], acc_sh.at[pl.ds(sid * SL + z * K, K)])
    rem = SL - (SL // K) * K
    if rem:
        pltpu.sync_copy(rows.at[0, pl.ds(0, rem)],
                        acc_sh.at[pl.ds(sid * SL + (SL // K) * K, rem)])
    plsc.subcore_barrier()

    def stage_meta(j, slot):
        base = tbase + j * K
        pltpu.async_copy(src_hbm.at[pl.ds(base, K)], srcc.at[slot],
                         msem.at[slot])
        pltpu.async_copy(dst_hbm.at[pl.ds(base, K)], dstc.at[slot],
                         msem.at[slot])
        pltpu.async_copy(ex_hbm.at[pl.ds(base, K)], exc.at[slot],
                         msem.at[slot])

    def wait_meta(slot):
        pltpu.make_async_copy(src_hbm.at[pl.ds(0, K)], srcc.at[slot],
                              msem.at[slot]).wait()
        pltpu.make_async_copy(dst_hbm.at[pl.ds(0, K)], dstc.at[slot],
                              msem.at[slot]).wait()
        pltpu.make_async_copy(ex_hbm.at[pl.ds(0, K)], exc.at[slot],
                              msem.at[slot]).wait()

    def copy_idx(src2d, mslot, dst2d, rslot):
        for u in range(K // L):
            dst2d[rslot, pl.ds(u * L, L)] = src2d[mslot, pl.ds(u * L, L)]

    def start_gather(rslot):
        pltpu.async_copy(h_hbm.at[gidx.at[rslot]], rows.at[rslot],
                         gsem.at[rslot])

    def wait_gather(rslot):
        pltpu.make_async_copy(h_hbm.at[gidx.at[rslot]], rows.at[rslot],
                              gsem.at[rslot]).wait()

    def start_scatter(rslot):
        pltpu.async_copy(rows.at[rslot], acc_sh.at[sidx.at[rslot]],
                         ssem.at[rslot], add=True)

    def wait_scatter(rslot):
        pltpu.make_async_copy(rows.at[rslot], acc_sh.at[sidx.at[rslot]],
                              ssem.at[rslot]).wait()

    def scale(mslot, rslot):
        @plsc.parallel_loop(0, K // 4, unroll=2)
        def _(r4):
            for k in range(4):
                ridx = jnp.full((L,), r4 * 4 + k, jnp.int32)
                wb = plsc.load_gather(exc.at[mslot], [ridx])
                for c in range(D // L):
                    cidx = c * L + lax.iota(jnp.int32, L)
                    v = plsc.load_gather(rows.at[rslot], [ridx, cidx])
                    plsc.store_scatter(rows.at[rslot], [ridx, cidx], v * wb)

    def section(s, b):
        j = 6 * s + b
        rs, rn = b % 3, (b + 1) % 3
        mn = (b + 1) % 6
        wait_gather(rs)
        if b <= 1:
            @pl.when(s > 0)
            def _():
                wait_scatter(rn)
        else:
            wait_scatter(rn)
        wait_meta(mn)
        copy_idx(srcc, mn, gidx, rn)
        start_gather(rn)
        scale(b, rs)
        copy_idx(dstc, b, sidx, rs)
        start_scatter(rs)
        stage_meta(jnp.minimum(j + 6, NCHUNK - 1), b)

    for b in range(6):
        stage_meta(b, b)
    wait_meta(0)
    copy_idx(srcc, 0, gidx, 0)
    start_gather(0)

    def superstep(s, _):
        for b in range(6):
            section(s, b)
        return _
    lax.fori_loop(0, NSS, superstep, None)

    wait_gather(0)
    wait_scatter(1)
    wait_scatter(2)
    for b in range(1, 6):
        wait_meta(b)

    plsc.subcore_barrier()
    pltpu.sync_copy(acc_sh.at[pl.ds(sid * SL, SL)],
                    out_hbm.at[cid, pl.ds(sid * SL, SL)])


_phase_b = functools.partial(
    pl.kernel,
    out_type=jax.ShapeDtypeStruct((NC, NP, D), jnp.float32),
    mesh=plsc.VectorSubcoreMesh(core_axis_name="c", subcore_axis_name="s"),
    compiler_params=_SC_PARAMS,
    scratch_types=[
        pltpu.VMEM((6, K), jnp.int32),
        pltpu.VMEM((6, K), jnp.int32),
        pltpu.VMEM((6, K), jnp.float32),
        pltpu.VMEM((3, K), jnp.int32),
        pltpu.VMEM((3, K), jnp.int32),
        pltpu.VMEM((3, K, D), jnp.float32),
        pltpu.VMEM_SHARED((NP, D), jnp.float32),
        pltpu.SemaphoreType.DMA((6,)),
        pltpu.SemaphoreType.DMA((3,)),
        pltpu.SemaphoreType.DMA((3,)),
    ],
)(_phase_b_body)


_RB = 512


def _dense1_body(x_ref, w_ref, asr_ref, adr_ref, h_ref, as_ref, ad_ref):
    h = jnp.dot(x_ref[...], w_ref[...], preferred_element_type=jnp.float32)
    h_ref[...] = h
    as_ref[...] = jnp.sum(h * asr_ref[...], axis=1, keepdims=True)
    ad_ref[...] = jnp.sum(h * adr_ref[...], axis=1, keepdims=True)


def _dense2_body(p_ref, den_ref, b_ref, w_ref, asr_ref, adr_ref,
                 h_ref, as_ref, ad_ref):
    i = pl.program_id(0)
    rows = i * _RB + lax.broadcasted_iota(jnp.int32, (_RB, D), 0)
    den = den_ref[0] + den_ref[1] + 1e-16
    x = (p_ref[0] + p_ref[1]) / den + b_ref[...]
    x = jnp.where(rows < N, x, 0.0)
    h = jnp.dot(x, w_ref[...], preferred_element_type=jnp.float32)
    h_ref[...] = h
    as_ref[...] = jnp.sum(h * asr_ref[...], axis=1, keepdims=True)
    ad_ref[...] = jnp.sum(h * adr_ref[...], axis=1, keepdims=True)


def _dense1(x, w, a_src, a_dst):
    return pl.pallas_call(
        _dense1_body,
        grid=(NP // _RB,),
        in_specs=[
            pl.BlockSpec((_RB, D), lambda i: (i, 0)),
            pl.BlockSpec((D, D), lambda i: (0, 0)),
            pl.BlockSpec((1, D), lambda i: (0, 0)),
            pl.BlockSpec((1, D), lambda i: (0, 0)),
        ],
        out_specs=[
            pl.BlockSpec((_RB, D), lambda i: (i, 0)),
            pl.BlockSpec((_RB, 1), lambda i: (i, 0)),
            pl.BlockSpec((_RB, 1), lambda i: (i, 0)),
        ],
        out_shape=[
            jax.ShapeDtypeStruct((NP, D), jnp.float32),
            jax.ShapeDtypeStruct((NP, 1), jnp.float32),
            jax.ShapeDtypeStruct((NP, 1), jnp.float32),
        ],
    )(x, w, a_src.reshape(1, D), a_dst.reshape(1, D))


def _dense2(p, den, b, w, a_src, a_dst):
    return pl.pallas_call(
        _dense2_body,
        grid=(NP // _RB,),
        in_specs=[
            pl.BlockSpec((NC, _RB, D), lambda i: (0, i, 0)),
            pl.BlockSpec((NC, _RB, 1), lambda i: (0, i, 0)),
            pl.BlockSpec((1, D), lambda i: (0, 0)),
            pl.BlockSpec((D, D), lambda i: (0, 0)),
            pl.BlockSpec((1, D), lambda i: (0, 0)),
            pl.BlockSpec((1, D), lambda i: (0, 0)),
        ],
        out_specs=[
            pl.BlockSpec((_RB, D), lambda i: (i, 0)),
            pl.BlockSpec((_RB, 1), lambda i: (i, 0)),
            pl.BlockSpec((_RB, 1), lambda i: (i, 0)),
        ],
        out_shape=[
            jax.ShapeDtypeStruct((NP, D), jnp.float32),
            jax.ShapeDtypeStruct((NP, 1), jnp.float32),
            jax.ShapeDtypeStruct((NP, 1), jnp.float32),
        ],
    )(p, den.reshape(NC, NP, 1), b.reshape(1, D), w,
      a_src.reshape(1, D), a_dst.reshape(1, D))


_PB = 400


def _pool_body(p_ref, den_ref, b_ref, batch_ref, out_ref, acc, cnt):
    i = pl.program_id(0)
    den = den_ref[0] + den_ref[1] + 1e-16
    x = (p_ref[0] + p_ref[1]) / den + b_ref[...]
    onehot = (batch_ref[...] ==
              lax.broadcasted_iota(jnp.int32, (_PB, NG), 1)).astype(jnp.float32)
    psum = lax.dot_general(onehot, x, (((0,), (0,)), ((), ())),
                           preferred_element_type=jnp.float32)
    pcnt = lax.dot_general(onehot, jnp.ones((_PB, 1), jnp.float32),
                           (((0,), (0,)), ((), ())),
                           preferred_element_type=jnp.float32)

    @pl.when(i == 0)
    def _():
        acc[...] = jnp.zeros_like(acc)
        cnt[...] = jnp.zeros_like(cnt)

    acc[...] += psum
    cnt[...] += pcnt

    @pl.when(i == N // _PB - 1)
    def _():
        out_ref[...] = acc[...] / jnp.maximum(cnt[...], 1.0)


def _pool(p, den, b, batch):
    return pl.pallas_call(
        _pool_body,
        grid=(N // _PB,),
        in_specs=[
            pl.BlockSpec((NC, _PB, D), lambda i: (0, i, 0)),
            pl.BlockSpec((NC, _PB, 1), lambda i: (0, i, 0)),
            pl.BlockSpec((1, D), lambda i: (0, 0)),
            pl.BlockSpec((_PB, 1), lambda i: (i, 0)),
        ],
        out_specs=pl.BlockSpec((NG, D), lambda i: (0, 0)),
        out_shape=jax.ShapeDtypeStruct((NG, D), jnp.float32),
        scratch_shapes=[
            pltpu.VMEM((NG, D), jnp.float32),
            pltpu.VMEM((NG, 1), jnp.float32),
        ],
    )(p, den.reshape(NC, NP, 1), b.reshape(1, D), batch.reshape(N, 1))


def kernel(x, edge_index, batch,
           W1, a_src1, a_dst1, b1, W2, a_src2, a_dst2, b2,
           W3, a_src3, a_dst3, b3):
    loop = jnp.arange(N, dtype=jnp.int32)
    padi = N + jnp.arange(E2P - E - N, dtype=jnp.int32) % (NP - N)
    src = jnp.concatenate([edge_index[0], loop, padi])
    dst = jnp.concatenate([edge_index[1], loop, padi])
    xp = jnp.pad(x, ((0, NP - N), (0, 0)))

    h, asv, adv = _dense1(xp, W1, a_src1, a_dst1)
    for (w, a_s, a_d, b) in ((W2, a_src2, a_dst2, b1),
                             (W3, a_src3, a_dst3, b2)):
        den, ex = _phase_a(src, dst, asv.reshape(NP), adv.reshape(NP))
        p = _phase_b(h, src, dst, ex)
        h, asv, adv = _dense2(p, den, b, w, a_s, a_d)
    den, ex = _phase_a(src, dst, asv.reshape(NP), adv.reshape(NP))
    p = _phase_b(h, src, dst, ex)
    return _pool(p, den, b3, batch)


# speedup vs baseline: 5.2860x; 1.0198x over previous
import functools

import jax
import jax.numpy as jnp
from jax import lax
from jax.experimental import pallas as pl
from jax.experimental.pallas import tpu as pltpu
from jax.experimental.pallas import tpu_sc as plsc

N = 10000
E = 320000
D = 128
NG = 128
NEG = 0.2

NC, NS, L = 2, 16, 16
NW = NC * NS
NP = 10240
K = 96
NCHUNK = 108
EPT = NCHUNK * K
E2P = NW * EPT
SL = NP // NS
NSS = NCHUNK // 6


def _leaky(v):
    return jnp.maximum(v, NEG * v)


def _gmax_of(as_v):
    def body(i, acc):
        return jnp.maximum(acc, as_v[pl.ds(i * L, L)])
    m = lax.fori_loop(0, NP // L, body, jnp.full((L,), -jnp.inf, jnp.float32))
    return jnp.max(m)


def _phase_a_body(src_hbm, dst_hbm, as_hbm, ad_hbm, den_hbm, ex_hbm,
                  as_v, ad_v, srcv, dstv, exv, dloc, accv, tmp2, dsh, sem):
    cid = lax.axis_index("c")
    sid = lax.axis_index("s")
    wid = cid * NS + sid
    c_as = pltpu.async_copy(as_hbm, as_v, sem.at[0])
    c_ad = pltpu.async_copy(ad_hbm, ad_v, sem.at[1])
    c_s = pltpu.async_copy(src_hbm.at[pl.ds(wid * EPT, EPT)], srcv, sem.at[2])
    c_d = pltpu.async_copy(dst_hbm.at[pl.ds(wid * EPT, EPT)], dstv, sem.at[3])

    def zero(i, _):
        dloc[pl.ds(i * L, L)] = jnp.zeros((L,), jnp.float32)
        return _
    lax.fori_loop(0, NP // L, zero, None)

    c_as.wait()
    gmax = _gmax_of(as_v)
    c_ad.wait()
    c_s.wait()
    c_d.wait()

    def edge(i, _):
        s = srcv[pl.ds(i * L, L)]
        d = dstv[pl.ds(i * L, L)]
        a1 = plsc.load_gather(as_v, [s])
        a2 = plsc.load_gather(ad_v, [d])
        e = _leaky(a1 + a2)
        m = _leaky(gmax + a2)
        ex = jnp.exp(e - m)
        exv[pl.ds(i * L, L)] = ex
        plsc.addupdate_scatter(dloc, [d], ex)
        return _
    lax.fori_loop(0, EPT // L, edge, None)
    c_ex = pltpu.async_copy(exv, ex_hbm.at[pl.ds(wid * EPT, EPT)], sem.at[0])

    pltpu.sync_copy(dloc, dsh.at[sid])
    plsc.subcore_barrier()
    pltpu.sync_copy(dsh.at[:, pl.ds(sid * SL, SL)], tmp2)

    def macc(i, _):
        tot = tmp2[0, pl.ds(i * L, L)]
        for t in range(1, NS):
            tot = tot + tmp2[t, pl.ds(i * L, L)]
        accv[pl.ds(i * L, L)] = tot
        return _
    lax.fori_loop(0, SL // L, macc, None)
    pltpu.sync_copy(accv, den_hbm.at[cid, pl.ds(sid * SL, SL)])
    c_ex.wait()


_SC_PARAMS = pltpu.CompilerParams(needs_layout_passes=False)

_phase_a = functools.partial(
    pl.kernel,
    out_type=(jax.ShapeDtypeStruct((NC, NP), jnp.float32),
              jax.ShapeDtypeStruct((E2P,), jnp.float32)),
    mesh=plsc.VectorSubcoreMesh(core_axis_name="c", subcore_axis_name="s"),
    compiler_params=_SC_PARAMS,
    scratch_types=[
        pltpu.VMEM((NP,), jnp.float32),
        pltpu.VMEM((NP,), jnp.float32),
        pltpu.VMEM((EPT,), jnp.int32),
        pltpu.VMEM((EPT,), jnp.int32),
        pltpu.VMEM((EPT,), jnp.float32),
        pltpu.VMEM((NP,), jnp.float32),
        pltpu.VMEM((SL,), jnp.float32),
        pltpu.VMEM((NS, SL), jnp.float32),
        pltpu.VMEM_SHARED((NS, NP), jnp.float32),
        pltpu.SemaphoreType.DMA((4,)),
    ],
)(_phase_a_body)


def _phase_b_body(h_hbm, src_hbm, dst_hbm, ex_hbm, out_hbm,
                  srcc, dstc, exc, gidx, sidx, rows, acc_sh,
                  msem, gsem, ssem):
    cid = lax.axis_index("c")
    sid = lax.axis_index("s")
    wid = cid * NS + sid
    tbase = wid * EPT

    def zrow(r, _):
        ridx = jnp.full((L,), r, jnp.int32)
        for c in range(D // L):
            cidx = c * L + lax.iota(jnp.int32, L)
            plsc.store_scatter(rows.at[0], [ridx, cidx],
                               jnp.zeros((L,), jnp.float32))
        return _
    lax.fori_loop(0, K, zrow, None)
    for z in range(SL // K):
        pltpu.sync_copy(rows.at[0], acc_sh.at[pl.ds(sid * SL + z * K, K)])
    rem = SL - (SL // K) * K
    if rem:
        pltpu.sync_copy(rows.at[0, pl.ds(0, rem)],
                        acc_sh.at[pl.ds(sid * SL + (SL // K) * K, rem)])
    plsc.subcore_barrier()

    def stage_meta(j, slot):
        base = tbase + j * K
        pltpu.async_copy(src_hbm.at[pl.ds(base, K)], srcc.at[slot],
                         msem.at[slot])
        pltpu.async_copy(dst_hbm.at[pl.ds(base, K)], dstc.at[slot],
                         msem.at[slot])
        pltpu.async_copy(ex_hbm.at[pl.ds(base, K)], exc.at[slot],
                         msem.at[slot])

    def wait_meta(slot):
        pltpu.make_async_copy(src_hbm.at[pl.ds(0, K)], srcc.at[slot],
                              msem.at[slot]).wait()
        pltpu.make_async_copy(dst_hbm.at[pl.ds(0, K)], dstc.at[slot],
                              msem.at[slot]).wait()
        pltpu.make_async_copy(ex_hbm.at[pl.ds(0, K)], exc.at[slot],
                              msem.at[slot]).wait()

    def copy_idx(src2d, mslot, dst2d, rslot):
        for u in range(K // L):
            dst2d[rslot, pl.ds(u * L, L)] = src2d[mslot, pl.ds(u * L, L)]

    def start_gather(rslot):
        pltpu.async_copy(h_hbm.at[gidx.at[rslot]], rows.at[rslot],
                         gsem.at[rslot])

    def wait_gather(rslot):
        pltpu.make_async_copy(h_hbm.at[gidx.at[rslot]], rows.at[rslot],
                              gsem.at[rslot]).wait()

    def start_scatter(rslot):
        pltpu.async_copy(rows.at[rslot], acc_sh.at[sidx.at[rslot]],
                         ssem.at[rslot], add=True)

    def wait_scatter(rslot):
        pltpu.make_async_copy(rows.at[rslot], acc_sh.at[sidx.at[rslot]],
                              ssem.at[rslot]).wait()

    def scale(mslot, rslot):
        @plsc.parallel_loop(0, K // 4, unroll=2)
        def _(r4):
            for k in range(4):
                ridx = jnp.full((L,), r4 * 4 + k, jnp.int32)
                wb = plsc.load_gather(exc.at[mslot], [ridx])
                for c in range(D // L):
                    cidx = c * L + lax.iota(jnp.int32, L)
                    v = plsc.load_gather(rows.at[rslot], [ridx, cidx])
                    plsc.store_scatter(rows.at[rslot], [ridx, cidx], v * wb)

    def section(s, b):
        j = 6 * s + b
        rs, rn = b % 3, (b + 1) % 3
        mn = (b + 1) % 6
        wait_gather(rs)
        if b <= 1:
            @pl.when(s > 0)
            def _():
                wait_scatter(rn)
        else:
            wait_scatter(rn)
        wait_meta(mn)
        copy_idx(srcc, mn, gidx, rn)
        start_gather(rn)
        scale(b, rs)
        copy_idx(dstc, b, sidx, rs)
        start_scatter(rs)
        stage_meta(jnp.minimum(j + 6, NCHUNK - 1), b)

    for b in range(6):
        stage_meta(b, b)
    wait_meta(0)
    copy_idx(srcc, 0, gidx, 0)
    start_gather(0)

    def superstep(s, _):
        for b in range(6):
            section(s, b)
        return _
    lax.fori_loop(0, NSS, superstep, None)

    wait_gather(0)
    wait_scatter(1)
    wait_scatter(2)
    for b in range(1, 6):
        wait_meta(b)

    plsc.subcore_barrier()
    pltpu.sync_copy(acc_sh.at[pl.ds(sid * SL, SL)],
                    out_hbm.at[cid, pl.ds(sid * SL, SL)])


_phase_b = functools.partial(
    pl.kernel,
    out_type=jax.ShapeDtypeStruct((NC, NP, D), jnp.float32),
    mesh=plsc.VectorSubcoreMesh(core_axis_name="c", subcore_axis_name="s"),
    compiler_params=_SC_PARAMS,
    scratch_types=[
        pltpu.VMEM((6, K), jnp.int32),
        pltpu.VMEM((6, K), jnp.int32),
        pltpu.VMEM((6, K), jnp.float32),
        pltpu.VMEM((3, K), jnp.int32),
        pltpu.VMEM((3, K), jnp.int32),
        pltpu.VMEM((3, K, D), jnp.float32),
        pltpu.VMEM_SHARED((NP, D), jnp.float32),
        pltpu.SemaphoreType.DMA((6,)),
        pltpu.SemaphoreType.DMA((3,)),
        pltpu.SemaphoreType.DMA((3,)),
    ],
)(_phase_b_body)


_RB = 512


def _dense1_body(x_ref, w_ref, asr_ref, adr_ref, h_ref, as_ref, ad_ref):
    i = pl.program_id(0)
    rows = i * _RB + lax.broadcasted_iota(jnp.int32, (_RB, D), 0)
    h = jnp.dot(x_ref[...], w_ref[...], preferred_element_type=jnp.float32)
    h = jnp.where(rows < N, h, 0.0)
    h_ref[...] = h
    as_ref[...] = jnp.sum(h * asr_ref[...], axis=1, keepdims=True)
    ad_ref[...] = jnp.sum(h * adr_ref[...], axis=1, keepdims=True)


def _dense2_body(p_ref, den_ref, b_ref, w_ref, asr_ref, adr_ref,
                 h_ref, as_ref, ad_ref):
    i = pl.program_id(0)
    rows = i * _RB + lax.broadcasted_iota(jnp.int32, (_RB, D), 0)
    den = den_ref[0] + den_ref[1] + 1e-16
    x = (p_ref[0] + p_ref[1]) / den + b_ref[...]
    x = jnp.where(rows < N, x, 0.0)
    h = jnp.dot(x, w_ref[...], preferred_element_type=jnp.float32)
    h_ref[...] = h
    as_ref[...] = jnp.sum(h * asr_ref[...], axis=1, keepdims=True)
    ad_ref[...] = jnp.sum(h * adr_ref[...], axis=1, keepdims=True)


def _dense1(x, w, a_src, a_dst):
    return pl.pallas_call(
        _dense1_body,
        grid=(NP // _RB,),
        in_specs=[
            pl.BlockSpec((_RB, D), lambda i: (i, 0)),
            pl.BlockSpec((D, D), lambda i: (0, 0)),
            pl.BlockSpec((1, D), lambda i: (0, 0)),
            pl.BlockSpec((1, D), lambda i: (0, 0)),
        ],
        out_specs=[
            pl.BlockSpec((_RB, D), lambda i: (i, 0)),
            pl.BlockSpec((_RB, 1), lambda i: (i, 0)),
            pl.BlockSpec((_RB, 1), lambda i: (i, 0)),
        ],
        out_shape=[
            jax.ShapeDtypeStruct((NP, D), jnp.float32),
            jax.ShapeDtypeStruct((NP, 1), jnp.float32),
            jax.ShapeDtypeStruct((NP, 1), jnp.float32),
        ],
    )(x, w, a_src.reshape(1, D), a_dst.reshape(1, D))


def _dense2(p, den, b, w, a_src, a_dst):
    return pl.pallas_call(
        _dense2_body,
        grid=(NP // _RB,),
        in_specs=[
            pl.BlockSpec((NC, _RB, D), lambda i: (0, i, 0)),
            pl.BlockSpec((NC, _RB, 1), lambda i: (0, i, 0)),
            pl.BlockSpec((1, D), lambda i: (0, 0)),
            pl.BlockSpec((D, D), lambda i: (0, 0)),
            pl.BlockSpec((1, D), lambda i: (0, 0)),
            pl.BlockSpec((1, D), lambda i: (0, 0)),
        ],
        out_specs=[
            pl.BlockSpec((_RB, D), lambda i: (i, 0)),
            pl.BlockSpec((_RB, 1), lambda i: (i, 0)),
            pl.BlockSpec((_RB, 1), lambda i: (i, 0)),
        ],
        out_shape=[
            jax.ShapeDtypeStruct((NP, D), jnp.float32),
            jax.ShapeDtypeStruct((NP, 1), jnp.float32),
            jax.ShapeDtypeStruct((NP, 1), jnp.float32),
        ],
    )(p, den.reshape(NC, NP, 1), b.reshape(1, D), w,
      a_src.reshape(1, D), a_dst.reshape(1, D))


_PB = 400


def _pool_body(p_ref, den_ref, b_ref, batch_ref, out_ref, acc, cnt):
    i = pl.program_id(0)
    den = den_ref[0] + den_ref[1] + 1e-16
    x = (p_ref[0] + p_ref[1]) / den + b_ref[...]
    onehot = (batch_ref[...] ==
              lax.broadcasted_iota(jnp.int32, (_PB, NG), 1)).astype(jnp.float32)
    psum = lax.dot_general(onehot, x, (((0,), (0,)), ((), ())),
                           preferred_element_type=jnp.float32)
    pcnt = lax.dot_general(onehot, jnp.ones((_PB, 1), jnp.float32),
                           (((0,), (0,)), ((), ())),
                           preferred_element_type=jnp.float32)

    @pl.when(i == 0)
    def _():
        acc[...] = jnp.zeros_like(acc)
        cnt[...] = jnp.zeros_like(cnt)

    acc[...] += psum
    cnt[...] += pcnt

    @pl.when(i == N // _PB - 1)
    def _():
        out_ref[...] = acc[...] / jnp.maximum(cnt[...], 1.0)


def _pool(p, den, b, batch):
    return pl.pallas_call(
        _pool_body,
        grid=(N // _PB,),
        in_specs=[
            pl.BlockSpec((NC, _PB, D), lambda i: (0, i, 0)),
            pl.BlockSpec((NC, _PB, 1), lambda i: (0, i, 0)),
            pl.BlockSpec((1, D), lambda i: (0, 0)),
            pl.BlockSpec((_PB, 1), lambda i: (i, 0)),
        ],
        out_specs=pl.BlockSpec((NG, D), lambda i: (0, 0)),
        out_shape=jax.ShapeDtypeStruct((NG, D), jnp.float32),
        scratch_shapes=[
            pltpu.VMEM((NG, D), jnp.float32),
            pltpu.VMEM((NG, 1), jnp.float32),
        ],
    )(p, den.reshape(NC, NP, 1), b.reshape(1, D), batch.reshape(N, 1))


def kernel(x, edge_index, batch,
           W1, a_src1, a_dst1, b1, W2, a_src2, a_dst2, b2,
           W3, a_src3, a_dst3, b3):
    loop = jnp.arange(N, dtype=jnp.int32)
    padi = N + jnp.arange(E2P - E - N, dtype=jnp.int32) % (NP - N)
    src = jnp.concatenate([edge_index[0], loop, padi])
    dst = jnp.concatenate([edge_index[1], loop, padi])

    h, asv, adv = _dense1(x, W1, a_src1, a_dst1)
    for (w, a_s, a_d, b) in ((W2, a_src2, a_dst2, b1),
                             (W3, a_src3, a_dst3, b2)):
        den, ex = _phase_a(src, dst, asv.reshape(NP), adv.reshape(NP))
        p = _phase_b(h, src, dst, ex)
        h, asv, adv = _dense2(p, den, b, w, a_s, a_d)
    den, ex = _phase_a(src, dst, asv.reshape(NP), adv.reshape(NP))
    p = _phase_b(h, src, dst, ex)
    return _pool(p, den, b3, batch)


# speedup vs baseline: 5.5218x; 1.0446x over previous
import functools

import jax
import jax.numpy as jnp
from jax import lax
from jax.experimental import pallas as pl
from jax.experimental.pallas import tpu as pltpu
from jax.experimental.pallas import tpu_sc as plsc

N = 10000
E = 320000
D = 128
NG = 128
NEG = 0.2

NC, NS, L = 2, 16, 16
NW = NC * NS
NP = 10240
K = 96
NCHUNK = 108
EPT = NCHUNK * K
E2P = NW * EPT
SL = NP // NS
NSS = NCHUNK // 6


def _leaky(v):
    return jnp.maximum(v, NEG * v)


def _gmax_of(as_v):
    def body(i, acc):
        return jnp.maximum(acc, as_v[pl.ds(i * L, L)])
    m = lax.fori_loop(0, NP // L, body, jnp.full((L,), -jnp.inf, jnp.float32))
    return jnp.max(m)


def _phase_a_body(src_hbm, dst_hbm, as_hbm, ad_hbm, den_hbm, ex_hbm,
                  as_v, ad_v, srcv, dstv, exv, dloc, accv, tmp2, dsh, sem):
    cid = lax.axis_index("c")
    sid = lax.axis_index("s")
    wid = cid * NS + sid
    c_as = pltpu.async_copy(as_hbm, as_v, sem.at[0])
    c_ad = pltpu.async_copy(ad_hbm, ad_v, sem.at[1])
    c_s = pltpu.async_copy(src_hbm.at[pl.ds(wid * EPT, EPT)], srcv, sem.at[2])
    c_d = pltpu.async_copy(dst_hbm.at[pl.ds(wid * EPT, EPT)], dstv, sem.at[3])

    def zero(i, _):
        dloc[pl.ds(i * L, L)] = jnp.zeros((L,), jnp.float32)
        return _
    lax.fori_loop(0, NP // L, zero, None)

    c_as.wait()
    gmax = _gmax_of(as_v)
    c_ad.wait()
    c_s.wait()
    c_d.wait()

    @plsc.parallel_loop(0, EPT // L, unroll=2)
    def edge(i):
        s = srcv[pl.ds(i * L, L)]
        d = dstv[pl.ds(i * L, L)]
        a1 = plsc.load_gather(as_v, [s])
        a2 = plsc.load_gather(ad_v, [d])
        e = _leaky(a1 + a2)
        m = _leaky(gmax + a2)
        ex = jnp.exp(e - m)
        exv[pl.ds(i * L, L)] = ex
        plsc.addupdate_scatter(dloc, [d], ex)
    c_ex = pltpu.async_copy(exv, ex_hbm.at[pl.ds(wid * EPT, EPT)], sem.at[0])

    pltpu.sync_copy(dloc, dsh.at[sid])
    plsc.subcore_barrier()
    pltpu.sync_copy(dsh.at[:, pl.ds(sid * SL, SL)], tmp2)

    def macc(i, _):
        tot = tmp2[0, pl.ds(i * L, L)]
        for t in range(1, NS):
            tot = tot + tmp2[t, pl.ds(i * L, L)]
        accv[pl.ds(i * L, L)] = tot
        return _
    lax.fori_loop(0, SL // L, macc, None)
    pltpu.sync_copy(accv, den_hbm.at[cid, pl.ds(sid * SL, SL)])
    c_ex.wait()


_SC_PARAMS = pltpu.CompilerParams(needs_layout_passes=False)

_phase_a = functools.partial(
    pl.kernel,
    out_type=(jax.ShapeDtypeStruct((NC, NP), jnp.float32),
              jax.ShapeDtypeStruct((E2P,), jnp.float32)),
    mesh=plsc.VectorSubcoreMesh(core_axis_name="c", subcore_axis_name="s"),
    compiler_params=_SC_PARAMS,
    scratch_types=[
        pltpu.VMEM((NP,), jnp.float32),
        pltpu.VMEM((NP,), jnp.float32),
        pltpu.VMEM((EPT,), jnp.int32),
        pltpu.VMEM((EPT,), jnp.int32),
        pltpu.VMEM((EPT,), jnp.float32),
        pltpu.VMEM((NP,), jnp.float32),
        pltpu.VMEM((SL,), jnp.float32),
        pltpu.VMEM((NS, SL), jnp.float32),
        pltpu.VMEM_SHARED((NS, NP), jnp.float32),
        pltpu.SemaphoreType.DMA((4,)),
    ],
)(_phase_a_body)


def _phase_b_body(h_hbm, src_hbm, dst_hbm, ex_hbm, out_hbm,
                  srcc, dstc, exc, gidx, sidx, rows, acc_sh,
                  msem, gsem, ssem):
    cid = lax.axis_index("c")
    sid = lax.axis_index("s")
    wid = cid * NS + sid
    tbase = wid * EPT

    def zrow(r, _):
        ridx = jnp.full((L,), r, jnp.int32)
        for c in range(D // L):
            cidx = c * L + lax.iota(jnp.int32, L)
            plsc.store_scatter(rows.at[0], [ridx, cidx],
                               jnp.zeros((L,), jnp.float32))
        return _
    lax.fori_loop(0, K, zrow, None)
    for z in range(SL // K):
        pltpu.sync_copy(rows.at[0], acc_sh.at[pl.ds(sid * SL + z * K, K)])
    rem = SL - (SL // K) * K
    if rem:
        pltpu.sync_copy(rows.at[0, pl.ds(0, rem)],
                        acc_sh.at[pl.ds(sid * SL + (SL // K) * K, rem)])
    plsc.subcore_barrier()

    def stage_meta(j, slot):
        base = tbase + j * K
        pltpu.async_copy(src_hbm.at[pl.ds(base, K)], srcc.at[slot],
                         msem.at[slot])
        pltpu.async_copy(dst_hbm.at[pl.ds(base, K)], dstc.at[slot],
                         msem.at[slot])
        pltpu.async_copy(ex_hbm.at[pl.ds(base, K)], exc.at[slot],
                         msem.at[slot])

    def wait_meta(slot):
        pltpu.make_async_copy(src_hbm.at[pl.ds(0, K)], srcc.at[slot],
                              msem.at[slot]).wait()
        pltpu.make_async_copy(dst_hbm.at[pl.ds(0, K)], dstc.at[slot],
                              msem.at[slot]).wait()
        pltpu.make_async_copy(ex_hbm.at[pl.ds(0, K)], exc.at[slot],
                              msem.at[slot]).wait()

    def copy_idx(src2d, mslot, dst2d, rslot):
        for u in range(K // L):
            dst2d[rslot, pl.ds(u * L, L)] = src2d[mslot, pl.ds(u * L, L)]

    def start_gather(rslot):
        pltpu.async_copy(h_hbm.at[gidx.at[rslot]], rows.at[rslot],
                         gsem.at[rslot])

    def wait_gather(rslot):
        pltpu.make_async_copy(h_hbm.at[gidx.at[rslot]], rows.at[rslot],
                              gsem.at[rslot]).wait()

    def start_scatter(rslot):
        pltpu.async_copy(rows.at[rslot], acc_sh.at[sidx.at[rslot]],
                         ssem.at[rslot], add=True)

    def wait_scatter(rslot):
        pltpu.make_async_copy(rows.at[rslot], acc_sh.at[sidx.at[rslot]],
                              ssem.at[rslot]).wait()

    def scale(mslot, rslot):
        @plsc.parallel_loop(0, K // 4, unroll=2)
        def _(r4):
            for k in range(4):
                ridx = jnp.full((L,), r4 * 4 + k, jnp.int32)
                wb = plsc.load_gather(exc.at[mslot], [ridx])
                for c in range(D // L):
                    cidx = c * L + lax.iota(jnp.int32, L)
                    v = plsc.load_gather(rows.at[rslot], [ridx, cidx])
                    plsc.store_scatter(rows.at[rslot], [ridx, cidx], v * wb)

    def section(s, b):
        j = 6 * s + b
        rs, rn = b % 3, (b + 1) % 3
        mn = (b + 1) % 6
        wait_gather(rs)
        if b <= 1:
            @pl.when(s > 0)
            def _():
                wait_scatter(rn)
        else:
            wait_scatter(rn)
        wait_meta(mn)
        copy_idx(srcc, mn, gidx, rn)
        start_gather(rn)
        scale(b, rs)
        copy_idx(dstc, b, sidx, rs)
        start_scatter(rs)
        stage_meta(jnp.minimum(j + 6, NCHUNK - 1), b)

    for b in range(6):
        stage_meta(b, b)
    wait_meta(0)
    copy_idx(srcc, 0, gidx, 0)
    start_gather(0)

    def superstep(s, _):
        for b in range(6):
            section(s, b)
        return _
    lax.fori_loop(0, NSS, superstep, None)

    wait_gather(0)
    wait_scatter(1)
    wait_scatter(2)
    for b in range(1, 6):
        wait_meta(b)

    plsc.subcore_barrier()
    pltpu.sync_copy(acc_sh.at[pl.ds(sid * SL, SL)],
                    out_hbm.at[cid, pl.ds(sid * SL, SL)])


_phase_b = functools.partial(
    pl.kernel,
    out_type=jax.ShapeDtypeStruct((NC, NP, D), jnp.float32),
    mesh=plsc.VectorSubcoreMesh(core_axis_name="c", subcore_axis_name="s"),
    compiler_params=_SC_PARAMS,
    scratch_types=[
        pltpu.VMEM((6, K), jnp.int32),
        pltpu.VMEM((6, K), jnp.int32),
        pltpu.VMEM((6, K), jnp.float32),
        pltpu.VMEM((3, K), jnp.int32),
        pltpu.VMEM((3, K), jnp.int32),
        pltpu.VMEM((3, K, D), jnp.float32),
        pltpu.VMEM_SHARED((NP, D), jnp.float32),
        pltpu.SemaphoreType.DMA((6,)),
        pltpu.SemaphoreType.DMA((3,)),
        pltpu.SemaphoreType.DMA((3,)),
    ],
)(_phase_b_body)


_RB = 512


def _dense1_body(x_ref, w_ref, asr_ref, adr_ref, h_ref, as_ref, ad_ref):
    i = pl.program_id(0)
    rows = i * _RB + lax.broadcasted_iota(jnp.int32, (_RB, D), 0)
    h = jnp.dot(x_ref[...], w_ref[...], preferred_element_type=jnp.float32)
    h = jnp.where(rows < N, h, 0.0)
    h_ref[...] = h
    as_ref[...] = jnp.sum(h * asr_ref[...], axis=1, keepdims=True)
    ad_ref[...] = jnp.sum(h * adr_ref[...], axis=1, keepdims=True)


def _dense2_body(p_ref, den_ref, b_ref, w_ref, asr_ref, adr_ref,
                 h_ref, as_ref, ad_ref):
    i = pl.program_id(0)
    rows = i * _RB + lax.broadcasted_iota(jnp.int32, (_RB, D), 0)
    den = den_ref[0] + den_ref[1] + 1e-16
    x = (p_ref[0] + p_ref[1]) / den + b_ref[...]
    x = jnp.where(rows < N, x, 0.0)
    h = jnp.dot(x, w_ref[...], preferred_element_type=jnp.float32)
    h_ref[...] = h
    as_ref[...] = jnp.sum(h * asr_ref[...], axis=1, keepdims=True)
    ad_ref[...] = jnp.sum(h * adr_ref[...], axis=1, keepdims=True)


def _dense1(x, w, a_src, a_dst):
    return pl.pallas_call(
        _dense1_body,
        grid=(NP // _RB,),
        in_specs=[
            pl.BlockSpec((_RB, D), lambda i: (i, 0)),
            pl.BlockSpec((D, D), lambda i: (0, 0)),
            pl.BlockSpec((1, D), lambda i: (0, 0)),
            pl.BlockSpec((1, D), lambda i: (0, 0)),
        ],
        out_specs=[
            pl.BlockSpec((_RB, D), lambda i: (i, 0)),
            pl.BlockSpec((_RB, 1), lambda i: (i, 0)),
            pl.BlockSpec((_RB, 1), lambda i: (i, 0)),
        ],
        out_shape=[
            jax.ShapeDtypeStruct((NP, D), jnp.float32),
            jax.ShapeDtypeStruct((NP, 1), jnp.float32),
            jax.ShapeDtypeStruct((NP, 1), jnp.float32),
        ],
    )(x, w, a_src.reshape(1, D), a_dst.reshape(1, D))


def _dense2(p, den, b, w, a_src, a_dst):
    return pl.pallas_call(
        _dense2_body,
        grid=(NP // _RB,),
        in_specs=[
            pl.BlockSpec((NC, _RB, D), lambda i: (0, i, 0)),
            pl.BlockSpec((NC, _RB, 1), lambda i: (0, i, 0)),
            pl.BlockSpec((1, D), lambda i: (0, 0)),
            pl.BlockSpec((D, D), lambda i: (0, 0)),
            pl.BlockSpec((1, D), lambda i: (0, 0)),
            pl.BlockSpec((1, D), lambda i: (0, 0)),
        ],
        out_specs=[
            pl.BlockSpec((_RB, D), lambda i: (i, 0)),
            pl.BlockSpec((_RB, 1), lambda i: (i, 0)),
            pl.BlockSpec((_RB, 1), lambda i: (i, 0)),
        ],
        out_shape=[
            jax.ShapeDtypeStruct((NP, D), jnp.float32),
            jax.ShapeDtypeStruct((NP, 1), jnp.float32),
            jax.ShapeDtypeStruct((NP, 1), jnp.float32),
        ],
    )(p, den.reshape(NC, NP, 1), b.reshape(1, D), w,
      a_src.reshape(1, D), a_dst.reshape(1, D))


_PB = 400


def _pool_body(p_ref, den_ref, b_ref, batch_ref, out_ref, acc, cnt):
    i = pl.program_id(0)
    den = den_ref[0] + den_ref[1] + 1e-16
    x = (p_ref[0] + p_ref[1]) / den + b_ref[...]
    onehot = (batch_ref[...] ==
              lax.broadcasted_iota(jnp.int32, (_PB, NG), 1)).astype(jnp.float32)
    psum = lax.dot_general(onehot, x, (((0,), (0,)), ((), ())),
                           preferred_element_type=jnp.float32)
    pcnt = lax.dot_general(onehot, jnp.ones((_PB, 1), jnp.float32),
                           (((0,), (0,)), ((), ())),
                           preferred_element_type=jnp.float32)

    @pl.when(i == 0)
    def _():
        acc[...] = jnp.zeros_like(acc)
        cnt[...] = jnp.zeros_like(cnt)

    acc[...] += psum
    cnt[...] += pcnt

    @pl.when(i == N // _PB - 1)
    def _():
        out_ref[...] = acc[...] / jnp.maximum(cnt[...], 1.0)


def _pool(p, den, b, batch):
    return pl.pallas_call(
        _pool_body,
        grid=(N // _PB,),
        in_specs=[
            pl.BlockSpec((NC, _PB, D), lambda i: (0, i, 0)),
            pl.BlockSpec((NC, _PB, 1), lambda i: (0, i, 0)),
            pl.BlockSpec((1, D), lambda i: (0, 0)),
            pl.BlockSpec((_PB, 1), lambda i: (i, 0)),
        ],
        out_specs=pl.BlockSpec((NG, D), lambda i: (0, 0)),
        out_shape=jax.ShapeDtypeStruct((NG, D), jnp.float32),
        scratch_shapes=[
            pltpu.VMEM((NG, D), jnp.float32),
            pltpu.VMEM((NG, 1), jnp.float32),
        ],
    )(p, den.reshape(NC, NP, 1), b.reshape(1, D), batch.reshape(N, 1))


def kernel(x, edge_index, batch,
           W1, a_src1, a_dst1, b1, W2, a_src2, a_dst2, b2,
           W3, a_src3, a_dst3, b3):
    loop = jnp.arange(N, dtype=jnp.int32)
    padi = N + jnp.arange(E2P - E - N, dtype=jnp.int32) % (NP - N)
    src = jnp.concatenate([edge_index[0], loop, padi])
    dst = jnp.concatenate([edge_index[1], loop, padi])

    h, asv, adv = _dense1(x, W1, a_src1, a_dst1)
    for (w, a_s, a_d, b) in ((W2, a_src2, a_dst2, b1),
                             (W3, a_src3, a_dst3, b2)):
        den, ex = _phase_a(src, dst, asv.reshape(NP), adv.reshape(NP))
        p = _phase_b(h, src, dst, ex)
        h, asv, adv = _dense2(p, den, b, w, a_s, a_d)
    den, ex = _phase_a(src, dst, asv.reshape(NP), adv.reshape(NP))
    p = _phase_b(h, src, dst, ex)
    return _pool(p, den, b3, batch)


# speedup vs baseline: 5.5295x; 1.0014x over previous
import functools

import jax
import jax.numpy as jnp
from jax import lax
from jax.experimental import pallas as pl
from jax.experimental.pallas import tpu as pltpu
from jax.experimental.pallas import tpu_sc as plsc

N = 10000
E = 320000
D = 128
NG = 128
NEG = 0.2

NC, NS, L = 2, 16, 16
NW = NC * NS
NP = 10240
K = 96
NCHUNK = 108
EPT = NCHUNK * K
E2P = NW * EPT
SL = NP // NS
NSS = NCHUNK // 6


def _leaky(v):
    return jnp.maximum(v, NEG * v)


def _gmax_of(as_v):
    def body(i, acc):
        return jnp.maximum(acc, as_v[pl.ds(i * L, L)])
    m = lax.fori_loop(0, NP // L, body, jnp.full((L,), -jnp.inf, jnp.float32))
    return jnp.max(m)


def _phase_a_body(src_hbm, dst_hbm, as_hbm, ad_hbm, den_hbm, ex_hbm,
                  as_v, ad_v, srcv, dstv, exv, dloc, accv, tmp2, dsh, sem):
    cid = lax.axis_index("c")
    sid = lax.axis_index("s")
    wid = cid * NS + sid
    c_as = pltpu.async_copy(as_hbm, as_v, sem.at[0])
    c_ad = pltpu.async_copy(ad_hbm, ad_v, sem.at[1])
    c_s = pltpu.async_copy(src_hbm.at[pl.ds(wid * EPT, EPT)], srcv, sem.at[2])
    c_d = pltpu.async_copy(dst_hbm.at[pl.ds(wid * EPT, EPT)], dstv, sem.at[3])

    def zero(i, _):
        dloc[pl.ds(i * L, L)] = jnp.zeros((L,), jnp.float32)
        return _
    lax.fori_loop(0, NP // L, zero, None)

    c_as.wait()
    gmax = _gmax_of(as_v)
    c_ad.wait()
    c_s.wait()
    c_d.wait()

    @plsc.parallel_loop(0, EPT // L, unroll=2)
    def edge(i):
        s = srcv[pl.ds(i * L, L)]
        d = dstv[pl.ds(i * L, L)]
        a1 = plsc.load_gather(as_v, [s])
        a2 = plsc.load_gather(ad_v, [d])
        e = _leaky(a1 + a2)
        m = _leaky(gmax + a2)
        ex = jnp.exp(e - m)
        exv[pl.ds(i * L, L)] = ex
        plsc.addupdate_scatter(dloc, [d], ex)
    c_ex = pltpu.async_copy(exv, ex_hbm.at[pl.ds(wid * EPT, EPT)], sem.at[0])

    pltpu.sync_copy(dloc, dsh.at[sid])
    plsc.subcore_barrier()
    pltpu.sync_copy(dsh.at[:, pl.ds(sid * SL, SL)], tmp2)

    def macc(i, _):
        tot = tmp2[0, pl.ds(i * L, L)]
        for t in range(1, NS):
            tot = tot + tmp2[t, pl.ds(i * L, L)]
        accv[pl.ds(i * L, L)] = tot
        return _
    lax.fori_loop(0, SL // L, macc, None)
    pltpu.sync_copy(accv, den_hbm.at[cid, pl.ds(sid * SL, SL)])
    c_ex.wait()


_SC_PARAMS = pltpu.CompilerParams(needs_layout_passes=False)

_phase_a = functools.partial(
    pl.kernel,
    out_type=(jax.ShapeDtypeStruct((NC, NP), jnp.float32),
              jax.ShapeDtypeStruct((E2P,), jnp.float32)),
    mesh=plsc.VectorSubcoreMesh(core_axis_name="c", subcore_axis_name="s"),
    compiler_params=_SC_PARAMS,
    scratch_types=[
        pltpu.VMEM((NP,), jnp.float32),
        pltpu.VMEM((NP,), jnp.float32),
        pltpu.VMEM((EPT,), jnp.int32),
        pltpu.VMEM((EPT,), jnp.int32),
        pltpu.VMEM((EPT,), jnp.float32),
        pltpu.VMEM((NP,), jnp.float32),
        pltpu.VMEM((SL,), jnp.float32),
        pltpu.VMEM((NS, SL), jnp.float32),
        pltpu.VMEM_SHARED((NS, NP), jnp.float32),
        pltpu.SemaphoreType.DMA((4,)),
    ],
)(_phase_a_body)


def _phase_b_body(h_hbm, src_hbm, dst_hbm, ex_hbm, out_hbm,
                  srcc, dstc, exc, gidx, sidx, rows, acc_sh,
                  msem, gsem, ssem):
    cid = lax.axis_index("c")
    sid = lax.axis_index("s")
    wid = cid * NS + sid
    tbase = wid * EPT

    def zrow(r, _):
        ridx = jnp.full((L,), r, jnp.int32)
        for c in range(D // L):
            cidx = c * L + lax.iota(jnp.int32, L)
            plsc.store_scatter(rows.at[0], [ridx, cidx],
                               jnp.zeros((L,), jnp.float32))
        return _
    lax.fori_loop(0, K, zrow, None)
    for z in range(SL // K):
        pltpu.sync_copy(rows.at[0], acc_sh.at[pl.ds(sid * SL + z * K, K)])
    rem = SL - (SL // K) * K
    if rem:
        pltpu.sync_copy(rows.at[0, pl.ds(0, rem)],
                        acc_sh.at[pl.ds(sid * SL + (SL // K) * K, rem)])
    plsc.subcore_barrier()

    def stage_meta(j, slot):
        base = tbase + j * K
        pltpu.async_copy(src_hbm.at[pl.ds(base, K)], srcc.at[slot],
                         msem.at[slot])
        pltpu.async_copy(dst_hbm.at[pl.ds(base, K)], dstc.at[slot],
                         msem.at[slot])
        pltpu.async_copy(ex_hbm.at[pl.ds(base, K)], exc.at[slot],
                         msem.at[slot])

    def wait_meta(slot):
        pltpu.make_async_copy(src_hbm.at[pl.ds(0, K)], srcc.at[slot],
                              msem.at[slot]).wait()
        pltpu.make_async_copy(dst_hbm.at[pl.ds(0, K)], dstc.at[slot],
                              msem.at[slot]).wait()
        pltpu.make_async_copy(ex_hbm.at[pl.ds(0, K)], exc.at[slot],
                              msem.at[slot]).wait()

    def copy_idx(src2d, mslot, dst2d, rslot):
        for u in range(K // L):
            dst2d[rslot, pl.ds(u * L, L)] = src2d[mslot, pl.ds(u * L, L)]

    def start_gather(rslot):
        pltpu.async_copy(h_hbm.at[gidx.at[rslot]], rows.at[rslot],
                         gsem.at[rslot])

    def wait_gather(rslot):
        pltpu.make_async_copy(h_hbm.at[gidx.at[rslot]], rows.at[rslot],
                              gsem.at[rslot]).wait()

    def start_scatter(rslot):
        pltpu.async_copy(rows.at[rslot], acc_sh.at[sidx.at[rslot]],
                         ssem.at[rslot], add=True)

    def wait_scatter(rslot):
        pltpu.make_async_copy(rows.at[rslot], acc_sh.at[sidx.at[rslot]],
                              ssem.at[rslot]).wait()

    def scale(mslot, rslot):
        @plsc.parallel_loop(0, K // 4, unroll=2)
        def _(r4):
            for k in range(4):
                ridx = jnp.full((L,), r4 * 4 + k, jnp.int32)
                wb = plsc.load_gather(exc.at[mslot], [ridx])
                for c in range(D // L):
                    cidx = c * L + lax.iota(jnp.int32, L)
                    v = plsc.load_gather(rows.at[rslot], [ridx, cidx])
                    plsc.store_scatter(rows.at[rslot], [ridx, cidx], v * wb)

    def section(s, b):
        j = 6 * s + b
        rs, rn = b % 3, (b + 1) % 3
        mn = (b + 1) % 6
        wait_gather(rs)
        if b <= 1:
            @pl.when(s > 0)
            def _():
                wait_scatter(rn)
        else:
            wait_scatter(rn)
        wait_meta(mn)
        copy_idx(srcc, mn, gidx, rn)
        start_gather(rn)
        scale(b, rs)
        copy_idx(dstc, b, sidx, rs)
        start_scatter(rs)
        stage_meta(jnp.minimum(j + 6, NCHUNK - 1), b)

    for b in range(6):
        stage_meta(b, b)
    wait_meta(0)
    copy_idx(srcc, 0, gidx, 0)
    start_gather(0)

    def superstep(s, _):
        for b in range(6):
            section(s, b)
        return _
    lax.fori_loop(0, NSS, superstep, None)

    wait_gather(0)
    wait_scatter(1)
    wait_scatter(2)
    for b in range(1, 6):
        wait_meta(b)

    plsc.subcore_barrier()
    pltpu.sync_copy(acc_sh.at[pl.ds(sid * SL, SL)],
                    out_hbm.at[cid, pl.ds(sid * SL, SL)])


_phase_b = functools.partial(
    pl.kernel,
    out_type=jax.ShapeDtypeStruct((NC, NP, D), jnp.float32),
    mesh=plsc.VectorSubcoreMesh(core_axis_name="c", subcore_axis_name="s"),
    compiler_params=_SC_PARAMS,
    scratch_types=[
        pltpu.VMEM((6, K), jnp.int32),
        pltpu.VMEM((6, K), jnp.int32),
        pltpu.VMEM((6, K), jnp.float32),
        pltpu.VMEM((3, K), jnp.int32),
        pltpu.VMEM((3, K), jnp.int32),
        pltpu.VMEM((3, K, D), jnp.float32),
        pltpu.VMEM_SHARED((NP, D), jnp.float32),
        pltpu.SemaphoreType.DMA((6,)),
        pltpu.SemaphoreType.DMA((3,)),
        pltpu.SemaphoreType.DMA((3,)),
    ],
)(_phase_b_body)


_RB = 512


def _dense1_body(x_ref, w_ref, asr_ref, adr_ref, h_ref, as_ref, ad_ref):
    i = pl.program_id(0)
    rows = i * _RB + lax.broadcasted_iota(jnp.int32, (_RB, D), 0)
    h = jnp.dot(x_ref[...], w_ref[...], preferred_element_type=jnp.float32)
    h = jnp.where(rows < N, h, 0.0)
    h_ref[...] = h
    as_ref[...] = jnp.sum(h * asr_ref[...], axis=1, keepdims=True)
    ad_ref[...] = jnp.sum(h * adr_ref[...], axis=1, keepdims=True)


def _dense2_body(p_ref, den_ref, b_ref, w_ref, asr_ref, adr_ref,
                 h_ref, as_ref, ad_ref):
    i = pl.program_id(0)
    rows = i * _RB + lax.broadcasted_iota(jnp.int32, (_RB, D), 0)
    den = den_ref[0] + den_ref[1]
    x = (p_ref[0] + p_ref[1]) / den + b_ref[...]
    x = jnp.where(rows < N, x, 0.0)
    h = jnp.dot(x, w_ref[...], preferred_element_type=jnp.float32)
    h_ref[...] = h
    as_ref[...] = jnp.sum(h * asr_ref[...], axis=1, keepdims=True)
    ad_ref[...] = jnp.sum(h * adr_ref[...], axis=1, keepdims=True)


def _dense1(x, w, a_src, a_dst):
    return pl.pallas_call(
        _dense1_body,
        grid=(NP // _RB,),
        in_specs=[
            pl.BlockSpec((_RB, D), lambda i: (i, 0)),
            pl.BlockSpec((D, D), lambda i: (0, 0)),
            pl.BlockSpec((1, D), lambda i: (0, 0)),
            pl.BlockSpec((1, D), lambda i: (0, 0)),
        ],
        out_specs=[
            pl.BlockSpec((_RB, D), lambda i: (i, 0)),
            pl.BlockSpec((_RB, 1), lambda i: (i, 0)),
            pl.BlockSpec((_RB, 1), lambda i: (i, 0)),
        ],
        out_shape=[
            jax.ShapeDtypeStruct((NP, D), jnp.float32),
            jax.ShapeDtypeStruct((NP, 1), jnp.float32),
            jax.ShapeDtypeStruct((NP, 1), jnp.float32),
        ],
    )(x, w, a_src.reshape(1, D), a_dst.reshape(1, D))


def _dense2(p, den, b, w, a_src, a_dst):
    return pl.pallas_call(
        _dense2_body,
        grid=(NP // _RB,),
        in_specs=[
            pl.BlockSpec((NC, _RB, D), lambda i: (0, i, 0)),
            pl.BlockSpec((NC, _RB, 1), lambda i: (0, i, 0)),
            pl.BlockSpec((1, D), lambda i: (0, 0)),
            pl.BlockSpec((D, D), lambda i: (0, 0)),
            pl.BlockSpec((1, D), lambda i: (0, 0)),
            pl.BlockSpec((1, D), lambda i: (0, 0)),
        ],
        out_specs=[
            pl.BlockSpec((_RB, D), lambda i: (i, 0)),
            pl.BlockSpec((_RB, 1), lambda i: (i, 0)),
            pl.BlockSpec((_RB, 1), lambda i: (i, 0)),
        ],
        out_shape=[
            jax.ShapeDtypeStruct((NP, D), jnp.float32),
            jax.ShapeDtypeStruct((NP, 1), jnp.float32),
            jax.ShapeDtypeStruct((NP, 1), jnp.float32),
        ],
    )(p, den.reshape(NC, NP, 1), b.reshape(1, D), w,
      a_src.reshape(1, D), a_dst.reshape(1, D))


_PB = 400


def _pool_body(p_ref, den_ref, b_ref, batch_ref, out_ref, acc, cnt):
    i = pl.program_id(0)
    den = den_ref[0] + den_ref[1]
    x = (p_ref[0] + p_ref[1]) / den + b_ref[...]
    onehot = (batch_ref[...] ==
              lax.broadcasted_iota(jnp.int32, (_PB, NG), 1)).astype(jnp.float32)
    psum = lax.dot_general(onehot, x, (((0,), (0,)), ((), ())),
                           preferred_element_type=jnp.float32)
    pcnt = lax.dot_general(onehot, jnp.ones((_PB, 1), jnp.float32),
                           (((0,), (0,)), ((), ())),
                           preferred_element_type=jnp.float32)

    @pl.when(i == 0)
    def _():
        acc[...] = jnp.zeros_like(acc)
        cnt[...] = jnp.zeros_like(cnt)

    acc[...] += psum
    cnt[...] += pcnt

    @pl.when(i == N // _PB - 1)
    def _():
        out_ref[...] = acc[...] / jnp.maximum(cnt[...], 1.0)


def _pool(p, den, b, batch):
    return pl.pallas_call(
        _pool_body,
        grid=(N // _PB,),
        in_specs=[
            pl.BlockSpec((NC, _PB, D), lambda i: (0, i, 0)),
            pl.BlockSpec((NC, _PB, 1), lambda i: (0, i, 0)),
            pl.BlockSpec((1, D), lambda i: (0, 0)),
            pl.BlockSpec((_PB, 1), lambda i: (i, 0)),
        ],
        out_specs=pl.BlockSpec((NG, D), lambda i: (0, 0)),
        out_shape=jax.ShapeDtypeStruct((NG, D), jnp.float32),
        scratch_shapes=[
            pltpu.VMEM((NG, D), jnp.float32),
            pltpu.VMEM((NG, 1), jnp.float32),
        ],
    )(p, den.reshape(NC, NP, 1), b.reshape(1, D), batch.reshape(N, 1))


def kernel(x, edge_index, batch,
           W1, a_src1, a_dst1, b1, W2, a_src2, a_dst2, b2,
           W3, a_src3, a_dst3, b3):
    loop = jnp.arange(N, dtype=jnp.int32)
    padi = N + jnp.arange(E2P - E - N, dtype=jnp.int32) % (NP - N)
    src = jnp.concatenate([edge_index[0], loop, padi])
    dst = jnp.concatenate([edge_index[1], loop, padi])

    h, asv, adv = _dense1(x, W1, a_src1, a_dst1)
    for (w, a_s, a_d, b) in ((W2, a_src2, a_dst2, b1),
                             (W3, a_src3, a_dst3, b2)):
        den, ex = _phase_a(src, dst, asv.reshape(NP), adv.reshape(NP))
        p = _phase_b(h, src, dst, ex)
        h, asv, adv = _dense2(p, den, b, w, a_s, a_d)
    den, ex = _phase_a(src, dst, asv.reshape(NP), adv.reshape(NP))
    p = _phase_b(h, src, dst, ex)
    return _pool(p, den, b3, batch)
